# Initial kernel scaffold; baseline (speedup 1.0000x reference)
import jax
import jax.numpy as jnp
from jax.experimental import pallas as pl


def kernel(x, edge_index, Wl0, bl0, Wr0, Wl1, bl1, Wr1, Wl2, bl2, Wr2, Wl3, bl3, Wr3):
    raise NotImplementedError("write your pallas kernel here")


# speedup vs baseline: 7.7674x; 7.7674x over previous
import functools

import jax
import jax.numpy as jnp
from jax import lax
from jax.experimental import pallas as pl
from jax.experimental.pallas import tpu as pltpu
from jax.experimental.pallas import tpu_sc as plsc

N = 10000
E = 320000
NP = 10240
NC, NS, L = 2, 16, 16
NW = NC * NS
CH = 128
NCHUNK = E // CH
EPW = E // NW
RPT = NP // NS

_MESH = plsc.VectorSubcoreMesh(
    core_axis_name="c", subcore_axis_name="s", num_cores=NC, num_subcores=NS
)
_SC_PARAMS = pltpu.CompilerParams(
    needs_layout_passes=False, use_tc_tiling_on_sc=False
)


def _seg64_body(src_hbm, dst_hbm, p_hbm, z_hbm, out_hbm,
                idx_s, idx_d, rows, obuf, acc_sh, gsem):
    c = lax.axis_index("c")
    s = lax.axis_index("s")
    wid = s * NC + c

    pltpu.sync_copy(z_hbm, obuf)
    pltpu.sync_copy(obuf, acc_sh.at[pl.ds(s * RPT, RPT), :])
    plsc.subcore_barrier()

    nch = (NCHUNK - 1 - wid) // NW + 1

    @pl.loop(0, nch)
    def _(i):
        base = (wid + i * NW) * CH
        pltpu.sync_copy(src_hbm.at[pl.ds(base, CH)], idx_s)
        pltpu.sync_copy(dst_hbm.at[pl.ds(base, CH)], idx_d)
        pltpu.async_copy(p_hbm.at[idx_s], rows, gsem).wait()
        pltpu.sync_copy(rows, acc_sh.at[idx_d], add=True)

    plsc.subcore_barrier()
    pltpu.sync_copy(acc_sh.at[pl.ds(s * RPT, RPT), :], obuf)
    pltpu.sync_copy(obuf, out_hbm.at[c, pl.ds(s * RPT, RPT), :])


_seg64 = pl.kernel(
    _seg64_body,
    out_type=jax.ShapeDtypeStruct((NC, NP, 64), jnp.float32),
    mesh=_MESH,
    scratch_types=[
        pltpu.VMEM((CH,), jnp.int32),
        pltpu.VMEM((CH,), jnp.int32),
        pltpu.VMEM((CH, 64), jnp.float32),
        pltpu.VMEM((RPT, 64), jnp.float32),
        pltpu.VMEM_SHARED((NP, 64), jnp.float32),
        pltpu.SemaphoreType.DMA,
    ],
    compiler_params=_SC_PARAMS,
)


def _w1_body(gather, *refs):
    if gather:
        src_hbm, dst_hbm, p_hbm, out_hbm, sidx_v, didx_v, pv, acc_v = refs
    else:
        dst_hbm, out_hbm, didx_v, acc_v = refs
    c = lax.axis_index("c")
    s = lax.axis_index("s")
    wid = s * NC + c

    @pl.loop(0, NP // L)
    def _(i):
        acc_v[pl.ds(i * L, L)] = jnp.zeros((L,), jnp.float32)

    pltpu.sync_copy(dst_hbm.at[pl.ds(wid * EPW, EPW)], didx_v)
    if gather:
        pltpu.sync_copy(src_hbm.at[pl.ds(wid * EPW, EPW)], sidx_v)
        pltpu.sync_copy(p_hbm, pv)
    ones = jnp.ones((L,), jnp.float32)

    @pl.loop(0, EPW // L)
    def _(i):
        d = didx_v[pl.ds(i * L, L)]
        if gather:
            sv = sidx_v[pl.ds(i * L, L)]
            v = plsc.load_gather(pv, [sv])
        else:
            v = ones
        plsc.addupdate_scatter(acc_v, [d], v)

    pltpu.sync_copy(acc_v, out_hbm.at[wid])


_cnt = pl.kernel(
    functools.partial(_w1_body, False),
    out_type=jax.ShapeDtypeStruct((NW, NP), jnp.float32),
    mesh=_MESH,
    scratch_types=[
        pltpu.VMEM((EPW,), jnp.int32),
        pltpu.VMEM((NP,), jnp.float32),
    ],
    compiler_params=_SC_PARAMS,
)

_seg1 = pl.kernel(
    functools.partial(_w1_body, True),
    out_type=jax.ShapeDtypeStruct((NW, NP), jnp.float32),
    mesh=_MESH,
    scratch_types=[
        pltpu.VMEM((EPW,), jnp.int32),
        pltpu.VMEM((EPW,), jnp.int32),
        pltpu.VMEM((NP,), jnp.float32),
        pltpu.VMEM((NP,), jnp.float32),
    ],
    compiler_params=_SC_PARAMS,
)


BR = 1000


def _pre_body(h_ref, wl_ref, bl_ref, wr_ref, p_ref, r_ref):
    h = h_ref[...]
    p_ref[...] = jnp.dot(h, wl_ref[...], preferred_element_type=jnp.float32)
    r_ref[...] = (
        jnp.dot(h, wr_ref[...], preferred_element_type=jnp.float32) + bl_ref[...]
    )


def _comb_body(a_ref, cp_ref, rp_ref, wl_ref, bl_ref, wr_ref, p_ref, r_ref):
    cnt = jnp.sum(cp_ref[...], axis=1, keepdims=True)
    inv = 1.0 / jnp.maximum(cnt, 1.0)
    h = jnp.maximum((a_ref[0] + a_ref[1]) * inv + rp_ref[...], 0.0)
    p_ref[...] = jnp.dot(h, wl_ref[...], preferred_element_type=jnp.float32)
    r_ref[...] = (
        jnp.dot(h, wr_ref[...], preferred_element_type=jnp.float32) + bl_ref[...]
    )


def _final_body(a3_ref, cp_ref, r3_ref, o_ref):
    cnt = jnp.sum(cp_ref[...], axis=1, keepdims=True)
    inv = 1.0 / jnp.maximum(cnt, 1.0)
    agg = jnp.sum(a3_ref[...], axis=1, keepdims=True)
    o_ref[...] = agg * inv + r3_ref[...]


def _pre(x, wl, bl, wr):
    din, dout = wl.shape
    return pl.pallas_call(
        _pre_body,
        grid=(N // BR,),
        in_specs=[
            pl.BlockSpec((BR, din), lambda i: (i, 0)),
            pl.BlockSpec((din, dout), lambda i: (0, 0)),
            pl.BlockSpec((1, dout), lambda i: (0, 0)),
            pl.BlockSpec((din, dout), lambda i: (0, 0)),
        ],
        out_specs=[pl.BlockSpec((BR, dout), lambda i: (i, 0))] * 2,
        out_shape=[jax.ShapeDtypeStruct((N, dout), jnp.float32)] * 2,
    )(x, wl, bl, wr)


def _comb(a, cp, rp, wl, bl, wr):
    din, dout = wl.shape
    return pl.pallas_call(
        _comb_body,
        grid=(N // BR,),
        in_specs=[
            pl.BlockSpec((NC, BR, 64), lambda i: (0, i, 0)),
            pl.BlockSpec((BR, NW), lambda i: (i, 0)),
            pl.BlockSpec((BR, din), lambda i: (i, 0)),
            pl.BlockSpec((din, dout), lambda i: (0, 0)),
            pl.BlockSpec((1, dout), lambda i: (0, 0)),
            pl.BlockSpec((din, dout), lambda i: (0, 0)),
        ],
        out_specs=[pl.BlockSpec((BR, dout), lambda i: (i, 0))] * 2,
        out_shape=[jax.ShapeDtypeStruct((N, dout), jnp.float32)] * 2,
    )(a, cp, rp, wl, bl, wr)


def _final(a3, cp, r3):
    return pl.pallas_call(
        _final_body,
        grid=(N // BR,),
        in_specs=[
            pl.BlockSpec((BR, NW), lambda i: (i, 0)),
            pl.BlockSpec((BR, NW), lambda i: (i, 0)),
            pl.BlockSpec((BR, 1), lambda i: (i, 0)),
        ],
        out_specs=pl.BlockSpec((BR, 1), lambda i: (i, 0)),
        out_shape=jax.ShapeDtypeStruct((N, 1), jnp.float32),
    )(a3, cp, r3)


def kernel(x, edge_index, Wl0, bl0, Wr0, Wl1, bl1, Wr1, Wl2, bl2, Wr2,
           Wl3, bl3, Wr3):
    src = edge_index[0]
    dst = edge_index[1]
    zrows = jnp.zeros((RPT, 64), jnp.float32)

    cntp = _cnt(dst)
    cp = cntp.T[:N]

    p, r = _pre(x, Wl0, bl0.reshape(1, -1), Wr0)
    for wl, bl, wr in ((Wl1, bl1, Wr1), (Wl2, bl2, Wr2), (Wl3, bl3, Wr3)):
        a = _seg64(src, dst, p, zrows)
        p, r = _comb(a, cp, r, wl, bl.reshape(1, -1), wr)

    p3 = jnp.pad(p.reshape(-1), (0, NP - N))
    a3 = _seg1(src, dst, p3)
    out = _final(a3.T[:N], cp, r)
    return out.reshape(-1)


# speedup vs baseline: 16.1374x; 2.0776x over previous
import functools

import jax
import jax.numpy as jnp
from jax import lax
from jax.experimental import pallas as pl
from jax.experimental.pallas import tpu as pltpu
from jax.experimental.pallas import tpu_sc as plsc

N = 10000
E = 320000
NP = 10240
NC, NS, L = 2, 16, 16
NW = NC * NS
CH = 128
NCHUNK = E // CH
EPW = E // NW
RPT = NP // NS

_MESH = plsc.VectorSubcoreMesh(
    core_axis_name="c", subcore_axis_name="s", num_cores=NC, num_subcores=NS
)
_SC_PARAMS = pltpu.CompilerParams(
    needs_layout_passes=False, use_tc_tiling_on_sc=False
)


CPW = NCHUNK // NW
K = 3
NWIN = CPW // K
NPAIR = NWIN // 2
CPT = RPT // CH


def _seg64_body(src_hbm, dst_hbm, p_hbm, z_hbm, out_hbm,
                sidx, didx, ra, rb, acc_sh, ga, gb):
    c = lax.axis_index("c")
    s = lax.axis_index("s")
    wid = s * NC + c
    row0 = wid * CPW

    pltpu.sync_copy(src_hbm.at[pl.ds(row0, CPW), :], sidx.at[pl.ds(0, CPW), :])
    pltpu.sync_copy(dst_hbm.at[pl.ds(row0, CPW), :], didx.at[pl.ds(0, CPW), :])
    @pl.when(wid < 4)
    def _():
        xrow = NW * CPW + wid
        pltpu.sync_copy(src_hbm.at[pl.ds(xrow, 1), :], sidx.at[pl.ds(CPW, 1), :])
        pltpu.sync_copy(dst_hbm.at[pl.ds(xrow, 1), :], didx.at[pl.ds(CPW, 1), :])

    pltpu.sync_copy(z_hbm, ra.at[0])
    for k in range(CPT):
        pltpu.sync_copy(ra.at[0], acc_sh.at[pl.ds(s * RPT + k * CH, CH), :])
    plsc.subcore_barrier()

    def fire(w, buf, sem):
        for t in range(K):
            pltpu.async_copy(p_hbm.at[sidx.at[w * K + t]], buf.at[t], sem)

    def drain(w, buf, sem):
        for t in range(K):
            pltpu.make_async_copy(p_hbm.at[sidx.at[w * K + t]], buf.at[t], sem).wait()

    def scat(w, buf):
        for t in range(K):
            pltpu.sync_copy(buf.at[t], acc_sh.at[didx.at[w * K + t]], add=True)

    fire(0, ra, ga)

    @pl.loop(0, NPAIR)
    def _(j):
        wa = 2 * j
        fire(wa + 1, rb, gb)
        drain(wa, ra, ga)
        scat(wa, ra)

        @pl.when(j < NPAIR - 1)
        def _():
            fire(wa + 2, ra, ga)

        drain(wa + 1, rb, gb)
        scat(wa + 1, rb)

    @pl.when(wid < 4)
    def _():
        pltpu.async_copy(p_hbm.at[sidx.at[CPW]], ra.at[0], ga).wait()
        pltpu.sync_copy(ra.at[0], acc_sh.at[didx.at[CPW]], add=True)

    plsc.subcore_barrier()
    for k in range(CPT):
        r = s * RPT + k * CH
        pltpu.sync_copy(acc_sh.at[pl.ds(r, CH), :], ra.at[0])
        pltpu.sync_copy(ra.at[0], out_hbm.at[c, pl.ds(r, CH), :])


_seg64 = pl.kernel(
    _seg64_body,
    out_type=jax.ShapeDtypeStruct((NC, NP, 64), jnp.float32),
    mesh=_MESH,
    scratch_types=[
        pltpu.VMEM((CPW + 1, CH), jnp.int32),
        pltpu.VMEM((CPW + 1, CH), jnp.int32),
        pltpu.VMEM((K, CH, 64), jnp.float32),
        pltpu.VMEM((K, CH, 64), jnp.float32),
        pltpu.VMEM_SHARED((NP, 64), jnp.float32),
        pltpu.SemaphoreType.DMA,
        pltpu.SemaphoreType.DMA,
    ],
    compiler_params=_SC_PARAMS,
)


def _w1_body(gather, *refs):
    if gather:
        src_hbm, dst_hbm, p_hbm, out_hbm, sidx_v, didx_v, pv, acc_v = refs
    else:
        dst_hbm, out_hbm, didx_v, acc_v = refs
    c = lax.axis_index("c")
    s = lax.axis_index("s")
    wid = s * NC + c

    @pl.loop(0, NP // L)
    def _(i):
        acc_v[pl.ds(i * L, L)] = jnp.zeros((L,), jnp.float32)

    pltpu.sync_copy(dst_hbm.at[pl.ds(wid * EPW, EPW)], didx_v)
    if gather:
        pltpu.sync_copy(src_hbm.at[pl.ds(wid * EPW, EPW)], sidx_v)
        pltpu.sync_copy(p_hbm, pv)
    ones = jnp.ones((L,), jnp.float32)

    @pl.loop(0, EPW // L)
    def _(i):
        d = didx_v[pl.ds(i * L, L)]
        if gather:
            sv = sidx_v[pl.ds(i * L, L)]
            v = plsc.load_gather(pv, [sv])
        else:
            v = ones
        plsc.addupdate_scatter(acc_v, [d], v)

    pltpu.sync_copy(acc_v, out_hbm.at[wid])


_cnt = pl.kernel(
    functools.partial(_w1_body, False),
    out_type=jax.ShapeDtypeStruct((NW, NP), jnp.float32),
    mesh=_MESH,
    scratch_types=[
        pltpu.VMEM((EPW,), jnp.int32),
        pltpu.VMEM((NP,), jnp.float32),
    ],
    compiler_params=_SC_PARAMS,
)

_seg1 = pl.kernel(
    functools.partial(_w1_body, True),
    out_type=jax.ShapeDtypeStruct((NW, NP), jnp.float32),
    mesh=_MESH,
    scratch_types=[
        pltpu.VMEM((EPW,), jnp.int32),
        pltpu.VMEM((EPW,), jnp.int32),
        pltpu.VMEM((NP,), jnp.float32),
        pltpu.VMEM((NP,), jnp.float32),
    ],
    compiler_params=_SC_PARAMS,
)


BR = 1000


def _pre_body(h_ref, wl_ref, bl_ref, wr_ref, p_ref, r_ref):
    h = h_ref[...]
    p_ref[...] = jnp.dot(h, wl_ref[...], preferred_element_type=jnp.float32)
    r_ref[...] = (
        jnp.dot(h, wr_ref[...], preferred_element_type=jnp.float32) + bl_ref[...]
    )


def _comb_body(a_ref, cp_ref, rp_ref, wl_ref, bl_ref, wr_ref, p_ref, r_ref):
    cnt = jnp.sum(cp_ref[...], axis=1, keepdims=True)
    inv = 1.0 / jnp.maximum(cnt, 1.0)
    h = jnp.maximum((a_ref[0] + a_ref[1]) * inv + rp_ref[...], 0.0)
    p_ref[...] = jnp.dot(h, wl_ref[...], preferred_element_type=jnp.float32)
    r_ref[...] = (
        jnp.dot(h, wr_ref[...], preferred_element_type=jnp.float32) + bl_ref[...]
    )


def _final_body(a3_ref, cp_ref, r3_ref, o_ref):
    cnt = jnp.sum(cp_ref[...], axis=1, keepdims=True)
    inv = 1.0 / jnp.maximum(cnt, 1.0)
    agg = jnp.sum(a3_ref[...], axis=1, keepdims=True)
    o_ref[...] = agg * inv + r3_ref[...]


def _pre(x, wl, bl, wr):
    din, dout = wl.shape
    return pl.pallas_call(
        _pre_body,
        grid=(N // BR,),
        in_specs=[
            pl.BlockSpec((BR, din), lambda i: (i, 0)),
            pl.BlockSpec((din, dout), lambda i: (0, 0)),
            pl.BlockSpec((1, dout), lambda i: (0, 0)),
            pl.BlockSpec((din, dout), lambda i: (0, 0)),
        ],
        out_specs=[pl.BlockSpec((BR, dout), lambda i: (i, 0))] * 2,
        out_shape=[jax.ShapeDtypeStruct((N, dout), jnp.float32)] * 2,
    )(x, wl, bl, wr)


def _comb(a, cp, rp, wl, bl, wr):
    din, dout = wl.shape
    return pl.pallas_call(
        _comb_body,
        grid=(N // BR,),
        in_specs=[
            pl.BlockSpec((NC, BR, 64), lambda i: (0, i, 0)),
            pl.BlockSpec((BR, NW), lambda i: (i, 0)),
            pl.BlockSpec((BR, din), lambda i: (i, 0)),
            pl.BlockSpec((din, dout), lambda i: (0, 0)),
            pl.BlockSpec((1, dout), lambda i: (0, 0)),
            pl.BlockSpec((din, dout), lambda i: (0, 0)),
        ],
        out_specs=[pl.BlockSpec((BR, dout), lambda i: (i, 0))] * 2,
        out_shape=[jax.ShapeDtypeStruct((N, dout), jnp.float32)] * 2,
    )(a, cp, rp, wl, bl, wr)


def _final(a3, cp, r3):
    return pl.pallas_call(
        _final_body,
        grid=(N // BR,),
        in_specs=[
            pl.BlockSpec((BR, NW), lambda i: (i, 0)),
            pl.BlockSpec((BR, NW), lambda i: (i, 0)),
            pl.BlockSpec((BR, 1), lambda i: (i, 0)),
        ],
        out_specs=pl.BlockSpec((BR, 1), lambda i: (i, 0)),
        out_shape=jax.ShapeDtypeStruct((N, 1), jnp.float32),
    )(a3, cp, r3)


def kernel(x, edge_index, Wl0, bl0, Wr0, Wl1, bl1, Wr1, Wl2, bl2, Wr2,
           Wl3, bl3, Wr3):
    src = edge_index[0]
    dst = edge_index[1]
    src2 = src.reshape(NCHUNK, CH)
    dst2 = dst.reshape(NCHUNK, CH)
    zrows = jnp.zeros((CH, 64), jnp.float32)

    cntp = _cnt(dst)
    cp = cntp.T[:N]

    p, r = _pre(x, Wl0, bl0.reshape(1, -1), Wr0)
    for wl, bl, wr in ((Wl1, bl1, Wr1), (Wl2, bl2, Wr2), (Wl3, bl3, Wr3)):
        a = _seg64(src2, dst2, p, zrows)
        p, r = _comb(a, cp, r, wl, bl.reshape(1, -1), wr)

    p3 = jnp.pad(p.reshape(-1), (0, NP - N))
    a3 = _seg1(src, dst, p3)
    out = _final(a3.T[:N], cp, r)
    return out.reshape(-1)


# speedup vs baseline: 16.4040x; 1.0165x over previous
import jax
import jax.numpy as jnp
from jax import lax
from jax.experimental import pallas as pl
from jax.experimental.pallas import tpu as pltpu
from jax.experimental.pallas import tpu_sc as plsc

N = 10000
E = 320000
NP = 10240
NC, NS, L = 2, 16, 16
NW = NC * NS
CH = 128
NCHUNK = E // CH
EPW = E // NW
RPT = NP // NS
CPW = NCHUNK // NW
K = 3
NWIN = CPW // K
NPAIR = NWIN // 2
CPT = RPT // CH

_MESH = plsc.VectorSubcoreMesh(
    core_axis_name="c", subcore_axis_name="s", num_cores=NC, num_subcores=NS
)
_SC_PARAMS = pltpu.CompilerParams(
    needs_layout_passes=False, use_tc_tiling_on_sc=False
)


def _seg_body(with_cnt, *refs):
    if with_cnt:
        (src_hbm, dst_hbm, p_hbm, z_hbm, z16_hbm, ii_hbm, out_hbm, outc_hbm,
         sidx, didx, ra, rb, cacc, iidx_v, acc_sh, red_sh, ga, gb) = refs
    else:
        (src_hbm, dst_hbm, p_hbm, z_hbm, out_hbm,
         sidx, didx, ra, rb, acc_sh, ga, gb) = refs
    c = lax.axis_index("c")
    s = lax.axis_index("s")
    wid = s * NC + c
    row0 = wid * CPW

    pltpu.sync_copy(src_hbm.at[pl.ds(row0, CPW), :], sidx.at[pl.ds(0, CPW), :])
    pltpu.sync_copy(dst_hbm.at[pl.ds(row0, CPW), :], didx.at[pl.ds(0, CPW), :])
    @pl.when(wid < 4)
    def _():
        xrow = NW * CPW + wid
        pltpu.sync_copy(src_hbm.at[pl.ds(xrow, 1), :], sidx.at[pl.ds(CPW, 1), :])
        pltpu.sync_copy(dst_hbm.at[pl.ds(xrow, 1), :], didx.at[pl.ds(CPW, 1), :])

    pltpu.sync_copy(z_hbm, ra.at[0])
    for k in range(CPT):
        pltpu.sync_copy(ra.at[0], acc_sh.at[pl.ds(s * RPT + k * CH, CH), :])
    if with_cnt:
        pltpu.sync_copy(z16_hbm, cacc)
        pltpu.sync_copy(ii_hbm, iidx_v)

        @pl.when(s == 0)
        def _():
            pltpu.sync_copy(cacc, red_sh)
    plsc.subcore_barrier()

    ones = jnp.ones((L,), jnp.float32)

    def fire(w, buf, sem):
        for t in range(K):
            pltpu.async_copy(p_hbm.at[sidx.at[w * K + t]], buf.at[t], sem)

    def drain(w, buf, sem):
        for t in range(K):
            pltpu.make_async_copy(p_hbm.at[sidx.at[w * K + t]], buf.at[t], sem).wait()

    def scat(w, buf):
        for t in range(K):
            pltpu.sync_copy(buf.at[t], acc_sh.at[didx.at[w * K + t]], add=True)

    def count(w):
        if not with_cnt:
            return
        for t in range(K):
            for u in range(CH // L):
                d = didx[w * K + t, pl.ds(u * L, L)]
                plsc.addupdate_scatter(
                    cacc, [jnp.right_shift(d, 4), jnp.bitwise_and(d, 15)], ones
                )

    fire(0, ra, ga)

    @pl.loop(0, NPAIR)
    def _(j):
        wa = 2 * j
        fire(wa + 1, rb, gb)
        count(wa)
        drain(wa, ra, ga)
        scat(wa, ra)

        @pl.when(j < NPAIR - 1)
        def _():
            fire(wa + 2, ra, ga)

        count(wa + 1)
        drain(wa + 1, rb, gb)
        scat(wa + 1, rb)

    @pl.when(wid < 4)
    def _():
        pltpu.async_copy(p_hbm.at[sidx.at[CPW]], ra.at[0], ga).wait()
        pltpu.sync_copy(ra.at[0], acc_sh.at[didx.at[CPW]], add=True)
        if with_cnt:
            for u in range(CH // L):
                d = didx[CPW, pl.ds(u * L, L)]
                plsc.addupdate_scatter(
                    cacc, [jnp.right_shift(d, 4), jnp.bitwise_and(d, 15)], ones
                )

    if with_cnt:
        for k in range(CPT):
            pltpu.sync_copy(cacc.at[pl.ds(k * CH, CH), :],
                            red_sh.at[iidx_v.at[k]], add=True)

    plsc.subcore_barrier()
    for k in range(CPT):
        r = s * RPT + k * CH
        pltpu.sync_copy(acc_sh.at[pl.ds(r, CH), :], ra.at[0])
        pltpu.sync_copy(ra.at[0], out_hbm.at[c, pl.ds(r, CH), :])
    if with_cnt:
        @pl.when(s == 0)
        def _():
            pltpu.sync_copy(red_sh, cacc)
            pltpu.sync_copy(cacc, outc_hbm.at[c])


def _make_seg(with_cnt):
    width = 64
    out_type = jax.ShapeDtypeStruct((NC, NP, width), jnp.float32)
    scratch = [
        pltpu.VMEM((CPW + 1, CH), jnp.int32),
        pltpu.VMEM((CPW + 1, CH), jnp.int32),
        pltpu.VMEM((K, CH, width), jnp.float32),
        pltpu.VMEM((K, CH, width), jnp.float32),
    ]
    if with_cnt:
        out_type = [out_type, jax.ShapeDtypeStruct((NC, RPT, L), jnp.float32)]
        scratch += [
            pltpu.VMEM((RPT, L), jnp.float32),
            pltpu.VMEM((CPT, CH), jnp.int32),
        ]
    scratch.append(pltpu.VMEM_SHARED((NP, width), jnp.float32))
    if with_cnt:
        scratch.append(pltpu.VMEM_SHARED((RPT, L), jnp.float32))
    scratch += [pltpu.SemaphoreType.DMA, pltpu.SemaphoreType.DMA]

    def body(*refs):
        return _seg_body(with_cnt, *refs)

    return pl.kernel(
        body,
        out_type=out_type,
        mesh=_MESH,
        scratch_types=scratch,
        compiler_params=_SC_PARAMS,
    )


_seg64c = _make_seg(True)
_seg64 = _make_seg(False)


def _seg1_body(src_hbm, dst_hbm, p_hbm, z_hbm, ii_hbm, out_hbm,
               sidx_v, didx_v, pv, acc_v, iidx_v, red_sh):
    c = lax.axis_index("c")
    s = lax.axis_index("s")
    wid = s * NC + c

    pltpu.sync_copy(z_hbm, acc_v)
    @pl.when(s == 0)
    def _():
        pltpu.sync_copy(acc_v, red_sh)
    pltpu.sync_copy(ii_hbm, iidx_v)
    pltpu.sync_copy(dst_hbm.at[pl.ds(wid * EPW, EPW)], didx_v)
    pltpu.sync_copy(src_hbm.at[pl.ds(wid * EPW, EPW)], sidx_v)
    pltpu.sync_copy(p_hbm, pv)
    plsc.subcore_barrier()

    @pl.loop(0, EPW // L)
    def _(i):
        d = didx_v[pl.ds(i * L, L)]
        sv = sidx_v[pl.ds(i * L, L)]
        v = plsc.load_gather(pv, [sv])
        plsc.addupdate_scatter(
            acc_v, [jnp.right_shift(d, 4), jnp.bitwise_and(d, 15)], v
        )

    for k in range(CPT):
        pltpu.sync_copy(acc_v.at[pl.ds(k * CH, CH), :],
                        red_sh.at[iidx_v.at[k]], add=True)
    plsc.subcore_barrier()

    @pl.when(s == 0)
    def _():
        pltpu.sync_copy(red_sh, acc_v)
        pltpu.sync_copy(acc_v, out_hbm.at[c])


_seg1 = pl.kernel(
    _seg1_body,
    out_type=jax.ShapeDtypeStruct((NC, RPT, L), jnp.float32),
    mesh=_MESH,
    scratch_types=[
        pltpu.VMEM((EPW,), jnp.int32),
        pltpu.VMEM((EPW,), jnp.int32),
        pltpu.VMEM((NP,), jnp.float32),
        pltpu.VMEM((RPT, L), jnp.float32),
        pltpu.VMEM((CPT, CH), jnp.int32),
        pltpu.VMEM_SHARED((RPT, L), jnp.float32),
    ],
    compiler_params=_SC_PARAMS,
)


BR = 1000


def _pre_body(h_ref, wl_ref, bl_ref, wr_ref, p_ref, r_ref):
    h = h_ref[...]
    p_ref[...] = jnp.dot(h, wl_ref[...], preferred_element_type=jnp.float32)
    r_ref[...] = (
        jnp.dot(h, wr_ref[...], preferred_element_type=jnp.float32) + bl_ref[...]
    )


def _comb1_body(a_ref, c0_ref, c1_ref, rp_ref, wl_ref, bl_ref, wr_ref,
                p_ref, r_ref, inv_ref):
    cnt = c0_ref[...] + c1_ref[...]
    inv = 1.0 / jnp.maximum(cnt, 1.0)
    h = jnp.maximum((a_ref[0] + a_ref[1]) * inv + rp_ref[...], 0.0)
    p_ref[...] = jnp.dot(h, wl_ref[...], preferred_element_type=jnp.float32)
    r_ref[...] = (
        jnp.dot(h, wr_ref[...], preferred_element_type=jnp.float32) + bl_ref[...]
    )
    inv_ref[...] = inv


def _comb_body(a_ref, inv_ref, rp_ref, wl_ref, bl_ref, wr_ref, p_ref, r_ref):
    h = jnp.maximum(
        (a_ref[0] + a_ref[1]) * inv_ref[...] + rp_ref[...], 0.0
    )
    p_ref[...] = jnp.dot(h, wl_ref[...], preferred_element_type=jnp.float32)
    r_ref[...] = (
        jnp.dot(h, wr_ref[...], preferred_element_type=jnp.float32) + bl_ref[...]
    )


def _final_body(a0_ref, a1_ref, inv_ref, r3_ref, o_ref):
    o_ref[...] = (a0_ref[...] + a1_ref[...]) * inv_ref[...] + r3_ref[...]


def _pre(x, wl, bl, wr):
    din, dout = wl.shape
    return pl.pallas_call(
        _pre_body,
        grid=(N // BR,),
        in_specs=[
            pl.BlockSpec((BR, din), lambda i: (i, 0)),
            pl.BlockSpec((din, dout), lambda i: (0, 0)),
            pl.BlockSpec((1, dout), lambda i: (0, 0)),
            pl.BlockSpec((din, dout), lambda i: (0, 0)),
        ],
        out_specs=[pl.BlockSpec((BR, dout), lambda i: (i, 0))] * 2,
        out_shape=[jax.ShapeDtypeStruct((N, dout), jnp.float32)] * 2,
    )(x, wl, bl, wr)


def _comb1(a, c0, c1, rp, wl, bl, wr):
    din, dout = wl.shape
    return pl.pallas_call(
        _comb1_body,
        grid=(N // BR,),
        in_specs=[
            pl.BlockSpec((NC, BR, 64), lambda i: (0, i, 0)),
            pl.BlockSpec((BR, 1), lambda i: (i, 0)),
            pl.BlockSpec((BR, 1), lambda i: (i, 0)),
            pl.BlockSpec((BR, din), lambda i: (i, 0)),
            pl.BlockSpec((din, dout), lambda i: (0, 0)),
            pl.BlockSpec((1, dout), lambda i: (0, 0)),
            pl.BlockSpec((din, dout), lambda i: (0, 0)),
        ],
        out_specs=[
            pl.BlockSpec((BR, dout), lambda i: (i, 0)),
            pl.BlockSpec((BR, dout), lambda i: (i, 0)),
            pl.BlockSpec((BR, 1), lambda i: (i, 0)),
        ],
        out_shape=[
            jax.ShapeDtypeStruct((N, dout), jnp.float32),
            jax.ShapeDtypeStruct((N, dout), jnp.float32),
            jax.ShapeDtypeStruct((N, 1), jnp.float32),
        ],
    )(a, c0, c1, rp, wl, bl, wr)


def _comb(a, inv, rp, wl, bl, wr):
    din, dout = wl.shape
    return pl.pallas_call(
        _comb_body,
        grid=(N // BR,),
        in_specs=[
            pl.BlockSpec((NC, BR, 64), lambda i: (0, i, 0)),
            pl.BlockSpec((BR, 1), lambda i: (i, 0)),
            pl.BlockSpec((BR, din), lambda i: (i, 0)),
            pl.BlockSpec((din, dout), lambda i: (0, 0)),
            pl.BlockSpec((1, dout), lambda i: (0, 0)),
            pl.BlockSpec((din, dout), lambda i: (0, 0)),
        ],
        out_specs=[pl.BlockSpec((BR, dout), lambda i: (i, 0))] * 2,
        out_shape=[jax.ShapeDtypeStruct((N, dout), jnp.float32)] * 2,
    )(a, inv, rp, wl, bl, wr)


def _final(a30, a31, inv, r3):
    spec = pl.BlockSpec((BR, 1), lambda i: (i, 0))
    return pl.pallas_call(
        _final_body,
        grid=(N // BR,),
        in_specs=[spec] * 4,
        out_specs=spec,
        out_shape=jax.ShapeDtypeStruct((N, 1), jnp.float32),
    )(a30, a31, inv, r3)


def kernel(x, edge_index, Wl0, bl0, Wr0, Wl1, bl1, Wr1, Wl2, bl2, Wr2,
           Wl3, bl3, Wr3):
    src = edge_index[0]
    dst = edge_index[1]
    src2 = src.reshape(NCHUNK, CH)
    dst2 = dst.reshape(NCHUNK, CH)
    z64 = jnp.zeros((CH, 64), jnp.float32)
    z16 = jnp.zeros((RPT, L), jnp.float32)
    iid = jnp.arange(RPT, dtype=jnp.int32).reshape(CPT, CH)

    p, r = _pre(x, Wl0, bl0.reshape(1, -1), Wr0)
    a, ac = _seg64c(src2, dst2, p, z64, z16, iid)
    acr = ac.reshape(NC, NP)
    c0 = acr[0, :N].reshape(N, 1)
    c1 = acr[1, :N].reshape(N, 1)
    p, r, inv = _comb1(a, c0, c1, r, Wl1, bl1.reshape(1, -1), Wr1)
    for wl, bl, wr in ((Wl2, bl2, Wr2), (Wl3, bl3, Wr3)):
        a = _seg64(src2, dst2, p, z64)
        p, r = _comb(a, inv, r, wl, bl.reshape(1, -1), wr)

    p3 = jnp.pad(p.reshape(-1), (0, NP - N))
    a3 = _seg1(src, dst, p3, z16, iid)
    a3r = a3.reshape(NC, NP)
    out = _final(a3r[0, :N].reshape(N, 1), a3r[1, :N].reshape(N, 1), inv, r)
    return out.reshape(-1)


# speedup vs baseline: 17.3017x; 1.0547x over previous
import jax
import jax.numpy as jnp
from jax import lax
from jax.experimental import pallas as pl
from jax.experimental.pallas import tpu as pltpu
from jax.experimental.pallas import tpu_sc as plsc

N = 10000
E = 320000
NP = 10240
NC, NS, L = 2, 16, 16
NW = NC * NS
CH = 128
NCHUNK = E // CH
EPW = E // NW
RPT = NP // NS
CPW = NCHUNK // NW
K = 3
NWIN = CPW // K
NPAIR = NWIN // 2
CPT = RPT // CH

_MESH = plsc.VectorSubcoreMesh(
    core_axis_name="c", subcore_axis_name="s", num_cores=NC, num_subcores=NS
)
_SC_PARAMS = pltpu.CompilerParams(
    needs_layout_passes=False, use_tc_tiling_on_sc=False
)


def _seg_body(with_cnt, *refs):
    if with_cnt:
        (ei_hbm, p_hbm, z_hbm, z16_hbm, ii_hbm, out_hbm, outc_hbm,
         eidx, ra, rb, cacc, iidx_v, acc_sh, red_sh, ga, gb, sa, sb) = refs
    else:
        (ei_hbm, p_hbm, z_hbm, out_hbm,
         eidx, ra, rb, acc_sh, ga, gb, sa, sb) = refs
    c = lax.axis_index("c")
    s = lax.axis_index("s")
    wid = s * NC + c
    row0 = wid * CPW

    pltpu.sync_copy(ei_hbm.at[pl.ds(row0, CPW)], eidx.at[pl.ds(0, CPW)])
    @pl.when(wid < 4)
    def _():
        xrow = NW * CPW + wid
        pltpu.sync_copy(ei_hbm.at[pl.ds(xrow, 1)], eidx.at[pl.ds(CPW, 1)])

    pltpu.sync_copy(z_hbm, ra.at[0])
    for k in range(CPT):
        pltpu.async_copy(ra.at[0], acc_sh.at[pl.ds(s * RPT + k * CH, CH), :], ga)
    for k in range(CPT):
        pltpu.make_async_copy(z_hbm, ra.at[0], ga).wait()
    if with_cnt:
        pltpu.sync_copy(z16_hbm, cacc)
        pltpu.sync_copy(ii_hbm, iidx_v)

        @pl.when(s == 0)
        def _():
            pltpu.sync_copy(cacc, red_sh)
    plsc.subcore_barrier()

    ones = jnp.ones((L,), jnp.float32)

    def fire(w, buf, sem):
        for t in range(K):
            pltpu.async_copy(p_hbm.at[eidx.at[w * K + t, 0]], buf.at[t], sem)

    def drain(w, buf, sem):
        for t in range(K):
            pltpu.make_async_copy(
                p_hbm.at[eidx.at[w * K + t, 0]], buf.at[t], sem).wait()

    def scat(w, buf, sem):
        for t in range(K):
            pltpu.async_copy(buf.at[t], acc_sh.at[eidx.at[w * K + t, 1]],
                             sem, add=True)

    def scat_drain(w, buf, sem):
        for t in range(K):
            pltpu.make_async_copy(buf.at[t], acc_sh.at[eidx.at[w * K + t, 1]],
                                  sem).wait()

    def count(w):
        if not with_cnt:
            return
        for t in range(K):
            for u in range(CH // L):
                d = eidx[w * K + t, 1, pl.ds(u * L, L)]
                plsc.addupdate_scatter(
                    cacc, [jnp.right_shift(d, 4), jnp.bitwise_and(d, 15)], ones
                )

    fire(0, ra, ga)

    @pl.loop(0, NPAIR)
    def _(j):
        wa = 2 * j

        @pl.when(j > 0)
        def _():
            scat_drain(wa - 1, rb, sb)

        fire(wa + 1, rb, gb)
        count(wa)
        drain(wa, ra, ga)
        scat(wa, ra, sa)
        count(wa + 1)
        drain(wa + 1, rb, gb)
        scat_drain(wa, ra, sa)

        @pl.when(j < NPAIR - 1)
        def _():
            fire(wa + 2, ra, ga)

        scat(wa + 1, rb, sb)

    scat_drain(2 * NPAIR - 1, rb, sb)

    @pl.when(wid < 4)
    def _():
        pltpu.async_copy(p_hbm.at[eidx.at[CPW, 0]], ra.at[0], ga).wait()
        pltpu.sync_copy(ra.at[0], acc_sh.at[eidx.at[CPW, 1]], add=True)
        if with_cnt:
            for u in range(CH // L):
                d = eidx[CPW, 1, pl.ds(u * L, L)]
                plsc.addupdate_scatter(
                    cacc, [jnp.right_shift(d, 4), jnp.bitwise_and(d, 15)], ones
                )

    if with_cnt:
        for k in range(CPT):
            pltpu.sync_copy(cacc.at[pl.ds(k * CH, CH), :],
                            red_sh.at[iidx_v.at[k]], add=True)

    plsc.subcore_barrier()
    bufs = [ra.at[0], ra.at[1], ra.at[2], rb.at[0], rb.at[1]]
    for k in range(CPT):
        pltpu.async_copy(acc_sh.at[pl.ds(s * RPT + k * CH, CH), :], bufs[k], ga)
    for k in range(CPT):
        pltpu.make_async_copy(acc_sh.at[pl.ds(s * RPT + k * CH, CH), :],
                              bufs[k], ga).wait()
    for k in range(CPT):
        pltpu.async_copy(bufs[k], out_hbm.at[c, pl.ds(s * RPT + k * CH, CH), :], ga)
    for k in range(CPT):
        pltpu.make_async_copy(bufs[k],
                              out_hbm.at[c, pl.ds(s * RPT + k * CH, CH), :],
                              ga).wait()
    if with_cnt:
        @pl.when(s == 0)
        def _():
            pltpu.sync_copy(red_sh, cacc)
            pltpu.sync_copy(cacc, outc_hbm.at[c])


def _make_seg(with_cnt):
    width = 64
    out_type = jax.ShapeDtypeStruct((NC, NP, width), jnp.float32)
    scratch = [
        pltpu.VMEM((CPW + 1, 2, CH), jnp.int32),
        pltpu.VMEM((K, CH, width), jnp.float32),
        pltpu.VMEM((K, CH, width), jnp.float32),
    ]
    if with_cnt:
        out_type = [out_type, jax.ShapeDtypeStruct((NC, RPT, L), jnp.float32)]
        scratch += [
            pltpu.VMEM((RPT, L), jnp.float32),
            pltpu.VMEM((CPT, CH), jnp.int32),
        ]
    scratch.append(pltpu.VMEM_SHARED((NP, width), jnp.float32))
    if with_cnt:
        scratch.append(pltpu.VMEM_SHARED((RPT, L), jnp.float32))
    scratch += [pltpu.SemaphoreType.DMA] * 4

    def body(*refs):
        return _seg_body(with_cnt, *refs)

    return pl.kernel(
        body,
        out_type=out_type,
        mesh=_MESH,
        scratch_types=scratch,
        compiler_params=_SC_PARAMS,
    )


_seg64c = _make_seg(True)
_seg64 = _make_seg(False)


def _seg1_body(ei_hbm, p_hbm, z_hbm, ii_hbm, out_hbm,
               eidx, pv, acc_v, iidx_v, red_sh):
    c = lax.axis_index("c")
    s = lax.axis_index("s")
    wid = s * NC + c
    row0 = wid * CPW

    pltpu.sync_copy(z_hbm, acc_v)
    @pl.when(s == 0)
    def _():
        pltpu.sync_copy(acc_v, red_sh)
    pltpu.sync_copy(ii_hbm, iidx_v)
    pltpu.sync_copy(ei_hbm.at[pl.ds(row0, CPW)], eidx.at[pl.ds(0, CPW)])
    @pl.when(wid < 4)
    def _():
        pltpu.sync_copy(ei_hbm.at[pl.ds(NW * CPW + wid, 1)],
                        eidx.at[pl.ds(CPW, 1)])
    pltpu.sync_copy(p_hbm, pv)
    plsc.subcore_barrier()

    def do_chunk(i):
        for u in range(CH // L):
            sv = eidx[i, 0, pl.ds(u * L, L)]
            d = eidx[i, 1, pl.ds(u * L, L)]
            v = plsc.load_gather(pv, [sv])
            plsc.addupdate_scatter(
                acc_v, [jnp.right_shift(d, 4), jnp.bitwise_and(d, 15)], v
            )

    @pl.loop(0, CPW)
    def _(i):
        do_chunk(i)

    @pl.when(wid < 4)
    def _():
        do_chunk(CPW)

    for k in range(CPT):
        pltpu.sync_copy(acc_v.at[pl.ds(k * CH, CH), :],
                        red_sh.at[iidx_v.at[k]], add=True)
    plsc.subcore_barrier()

    @pl.when(s == 0)
    def _():
        pltpu.sync_copy(red_sh, acc_v)
        pltpu.sync_copy(acc_v, out_hbm.at[c])


_seg1 = pl.kernel(
    _seg1_body,
    out_type=jax.ShapeDtypeStruct((NC, RPT, L), jnp.float32),
    mesh=_MESH,
    scratch_types=[
        pltpu.VMEM((CPW + 1, 2, CH), jnp.int32),
        pltpu.VMEM((NP,), jnp.float32),
        pltpu.VMEM((RPT, L), jnp.float32),
        pltpu.VMEM((CPT, CH), jnp.int32),
        pltpu.VMEM_SHARED((RPT, L), jnp.float32),
    ],
    compiler_params=_SC_PARAMS,
)


BR = 2000


def _pre_body(h_ref, wl_ref, bl_ref, wr_ref, p_ref, r_ref):
    h = h_ref[...]
    p_ref[...] = jnp.dot(h, wl_ref[...], preferred_element_type=jnp.float32)
    r_ref[...] = (
        jnp.dot(h, wr_ref[...], preferred_element_type=jnp.float32) + bl_ref[...]
    )


def _comb1_body(a_ref, c0_ref, c1_ref, rp_ref, wl_ref, bl_ref, wr_ref,
                p_ref, r_ref, inv_ref):
    cnt = c0_ref[...] + c1_ref[...]
    inv = 1.0 / jnp.maximum(cnt, 1.0)
    h = jnp.maximum((a_ref[0] + a_ref[1]) * inv + rp_ref[...], 0.0)
    p_ref[...] = jnp.dot(h, wl_ref[...], preferred_element_type=jnp.float32)
    r_ref[...] = (
        jnp.dot(h, wr_ref[...], preferred_element_type=jnp.float32) + bl_ref[...]
    )
    inv_ref[...] = inv


def _comb_body(a_ref, inv_ref, rp_ref, wl_ref, bl_ref, wr_ref, p_ref, r_ref):
    h = jnp.maximum(
        (a_ref[0] + a_ref[1]) * inv_ref[...] + rp_ref[...], 0.0
    )
    p_ref[...] = jnp.dot(h, wl_ref[...], preferred_element_type=jnp.float32)
    r_ref[...] = (
        jnp.dot(h, wr_ref[...], preferred_element_type=jnp.float32) + bl_ref[...]
    )


def _final_body(a0_ref, a1_ref, inv_ref, r3_ref, o_ref):
    o_ref[...] = (a0_ref[...] + a1_ref[...]) * inv_ref[...] + r3_ref[...]


def _pre(x, wl, bl, wr):
    din, dout = wl.shape
    return pl.pallas_call(
        _pre_body,
        grid=(N // BR,),
        in_specs=[
            pl.BlockSpec((BR, din), lambda i: (i, 0)),
            pl.BlockSpec((din, dout), lambda i: (0, 0)),
            pl.BlockSpec((1, dout), lambda i: (0, 0)),
            pl.BlockSpec((din, dout), lambda i: (0, 0)),
        ],
        out_specs=[pl.BlockSpec((BR, dout), lambda i: (i, 0))] * 2,
        out_shape=[jax.ShapeDtypeStruct((N, dout), jnp.float32)] * 2,
    )(x, wl, bl, wr)


def _comb1(a, c0, c1, rp, wl, bl, wr):
    din, dout = wl.shape
    return pl.pallas_call(
        _comb1_body,
        grid=(N // BR,),
        in_specs=[
            pl.BlockSpec((NC, BR, 64), lambda i: (0, i, 0)),
            pl.BlockSpec((BR, 1), lambda i: (i, 0)),
            pl.BlockSpec((BR, 1), lambda i: (i, 0)),
            pl.BlockSpec((BR, din), lambda i: (i, 0)),
            pl.BlockSpec((din, dout), lambda i: (0, 0)),
            pl.BlockSpec((1, dout), lambda i: (0, 0)),
            pl.BlockSpec((din, dout), lambda i: (0, 0)),
        ],
        out_specs=[
            pl.BlockSpec((BR, dout), lambda i: (i, 0)),
            pl.BlockSpec((BR, dout), lambda i: (i, 0)),
            pl.BlockSpec((BR, 1), lambda i: (i, 0)),
        ],
        out_shape=[
            jax.ShapeDtypeStruct((N, dout), jnp.float32),
            jax.ShapeDtypeStruct((N, dout), jnp.float32),
            jax.ShapeDtypeStruct((N, 1), jnp.float32),
        ],
    )(a, c0, c1, rp, wl, bl, wr)


def _comb(a, inv, rp, wl, bl, wr):
    din, dout = wl.shape
    return pl.pallas_call(
        _comb_body,
        grid=(N // BR,),
        in_specs=[
            pl.BlockSpec((NC, BR, 64), lambda i: (0, i, 0)),
            pl.BlockSpec((BR, 1), lambda i: (i, 0)),
            pl.BlockSpec((BR, din), lambda i: (i, 0)),
            pl.BlockSpec((din, dout), lambda i: (0, 0)),
            pl.BlockSpec((1, dout), lambda i: (0, 0)),
            pl.BlockSpec((din, dout), lambda i: (0, 0)),
        ],
        out_specs=[pl.BlockSpec((BR, dout), lambda i: (i, 0))] * 2,
        out_shape=[jax.ShapeDtypeStruct((N, dout), jnp.float32)] * 2,
    )(a, inv, rp, wl, bl, wr)


def _final(a30, a31, inv, r3):
    spec = pl.BlockSpec((BR, 1), lambda i: (i, 0))
    return pl.pallas_call(
        _final_body,
        grid=(N // BR,),
        in_specs=[spec] * 4,
        out_specs=spec,
        out_shape=jax.ShapeDtypeStruct((N, 1), jnp.float32),
    )(a30, a31, inv, r3)


def kernel(x, edge_index, Wl0, bl0, Wr0, Wl1, bl1, Wr1, Wl2, bl2, Wr2,
           Wl3, bl3, Wr3):
    ei3 = edge_index.reshape(2, NCHUNK, CH).transpose(1, 0, 2)
    z64 = jnp.zeros((CH, 64), jnp.float32)
    z16 = jnp.zeros((RPT, L), jnp.float32)
    iid = jnp.arange(RPT, dtype=jnp.int32).reshape(CPT, CH)

    p, r = _pre(x, Wl0, bl0.reshape(1, -1), Wr0)
    a, ac = _seg64c(ei3, p, z64, z16, iid)
    acr = ac.reshape(NC, NP)
    c0 = acr[0, :N].reshape(N, 1)
    c1 = acr[1, :N].reshape(N, 1)
    p, r, inv = _comb1(a, c0, c1, r, Wl1, bl1.reshape(1, -1), Wr1)
    for wl, bl, wr in ((Wl2, bl2, Wr2), (Wl3, bl3, Wr3)):
        a = _seg64(ei3, p, z64)
        p, r = _comb(a, inv, r, wl, bl.reshape(1, -1), wr)

    p3 = jnp.pad(p.reshape(-1), (0, NP - N))
    a3 = _seg1(ei3, p3, z16, iid)
    a3r = a3.reshape(NC, NP)
    out = _final(a3r[0, :N].reshape(N, 1), a3r[1, :N].reshape(N, 1), inv, r)
    return out.reshape(-1)


# speedup vs baseline: 17.3154x; 1.0008x over previous
import jax
import jax.numpy as jnp
from jax import lax
from jax.experimental import pallas as pl
from jax.experimental.pallas import tpu as pltpu
from jax.experimental.pallas import tpu_sc as plsc

N = 10000
E = 320000
NP = 10240
NC, NS, L = 2, 16, 16
NW = NC * NS
CH = 128
NCHUNK = E // CH
EPW = E // NW
RPT = NP // NS
CPW = NCHUNK // NW
K = 3
NWIN = CPW // K
NPAIR = NWIN // 2
CPT = RPT // CH

_MESH = plsc.VectorSubcoreMesh(
    core_axis_name="c", subcore_axis_name="s", num_cores=NC, num_subcores=NS
)
_SC_PARAMS = pltpu.CompilerParams(
    needs_layout_passes=False, use_tc_tiling_on_sc=False
)


def _seg_body(with_cnt, *refs):
    if with_cnt:
        (ei_hbm, p_hbm, z_hbm, z16_hbm, ii_hbm, out_hbm, outc_hbm,
         eidx, ra, rb, cacc, iidx_v, acc_sh, red_sh, ga, gb, sa, sb) = refs
    else:
        (ei_hbm, p_hbm, z_hbm, out_hbm,
         eidx, ra, rb, acc_sh, ga, gb, sa, sb) = refs
    c = lax.axis_index("c")
    s = lax.axis_index("s")
    wid = s * NC + c
    row0 = wid * CPW

    pltpu.sync_copy(ei_hbm.at[pl.ds(row0, CPW)], eidx.at[pl.ds(0, CPW)])
    @pl.when(wid < 4)
    def _():
        xrow = NW * CPW + wid
        pltpu.sync_copy(ei_hbm.at[pl.ds(xrow, 1)], eidx.at[pl.ds(CPW, 1)])

    pltpu.sync_copy(z_hbm, ra.at[0])
    for k in range(CPT):
        pltpu.async_copy(ra.at[0], acc_sh.at[pl.ds(s * RPT + k * CH, CH), :], ga)
    for k in range(CPT):
        pltpu.make_async_copy(z_hbm, ra.at[0], ga).wait()
    if with_cnt:
        pltpu.sync_copy(z16_hbm, cacc)
        pltpu.sync_copy(ii_hbm, iidx_v)

        @pl.when(s == 0)
        def _():
            pltpu.sync_copy(cacc, red_sh)
    plsc.subcore_barrier()

    ones = jnp.ones((L,), jnp.float32)

    def fire(w, buf, sem):
        for t in range(K):
            pltpu.async_copy(p_hbm.at[eidx.at[w * K + t, 0]], buf.at[t], sem)

    def drain(w, buf, sem):
        for t in range(K):
            pltpu.make_async_copy(
                p_hbm.at[eidx.at[w * K + t, 0]], buf.at[t], sem).wait()

    def scat(w, buf, sem):
        for t in range(K):
            pltpu.async_copy(buf.at[t], acc_sh.at[eidx.at[w * K + t, 1]],
                             sem, add=True)

    def scat_drain(w, buf, sem):
        for t in range(K):
            pltpu.make_async_copy(buf.at[t], acc_sh.at[eidx.at[w * K + t, 1]],
                                  sem).wait()

    def count(w):
        if not with_cnt:
            return
        for t in range(K):
            for u in range(CH // L):
                d = eidx[w * K + t, 1, pl.ds(u * L, L)]
                plsc.addupdate_scatter(
                    cacc, [jnp.right_shift(d, 4), jnp.bitwise_and(d, 15)], ones
                )

    fire(0, ra, ga)

    @pl.loop(0, NPAIR)
    def _(j):
        wa = 2 * j

        @pl.when(j > 0)
        def _():
            scat_drain(wa - 1, rb, sb)

        fire(wa + 1, rb, gb)
        count(wa)
        drain(wa, ra, ga)
        scat(wa, ra, sa)
        count(wa + 1)
        drain(wa + 1, rb, gb)
        scat_drain(wa, ra, sa)

        @pl.when(j < NPAIR - 1)
        def _():
            fire(wa + 2, ra, ga)

        scat(wa + 1, rb, sb)

    scat_drain(2 * NPAIR - 1, rb, sb)

    @pl.when(wid < 4)
    def _():
        pltpu.async_copy(p_hbm.at[eidx.at[CPW, 0]], ra.at[0], ga).wait()
        pltpu.sync_copy(ra.at[0], acc_sh.at[eidx.at[CPW, 1]], add=True)
        if with_cnt:
            for u in range(CH // L):
                d = eidx[CPW, 1, pl.ds(u * L, L)]
                plsc.addupdate_scatter(
                    cacc, [jnp.right_shift(d, 4), jnp.bitwise_and(d, 15)], ones
                )

    if with_cnt:
        for k in range(CPT):
            pltpu.sync_copy(cacc.at[pl.ds(k * CH, CH), :],
                            red_sh.at[iidx_v.at[k]], add=True)

    plsc.subcore_barrier()
    bufs = [ra.at[0], ra.at[1], ra.at[2], rb.at[0], rb.at[1]]
    for k in range(CPT):
        pltpu.async_copy(acc_sh.at[pl.ds(s * RPT + k * CH, CH), :], bufs[k], ga)
    for k in range(CPT):
        pltpu.make_async_copy(acc_sh.at[pl.ds(s * RPT + k * CH, CH), :],
                              bufs[k], ga).wait()
    for k in range(CPT):
        pltpu.async_copy(bufs[k], out_hbm.at[c, pl.ds(s * RPT + k * CH, CH), :], ga)
    for k in range(CPT):
        pltpu.make_async_copy(bufs[k],
                              out_hbm.at[c, pl.ds(s * RPT + k * CH, CH), :],
                              ga).wait()
    if with_cnt:
        @pl.when(s == 0)
        def _():
            pltpu.sync_copy(red_sh, cacc)
            pltpu.sync_copy(cacc, outc_hbm.at[c])


def _make_seg(with_cnt):
    width = 64
    out_type = jax.ShapeDtypeStruct((NC, NP, width), jnp.float32)
    scratch = [
        pltpu.VMEM((CPW + 1, 2, CH), jnp.int32),
        pltpu.VMEM((K, CH, width), jnp.float32),
        pltpu.VMEM((K, CH, width), jnp.float32),
    ]
    if with_cnt:
        out_type = [out_type, jax.ShapeDtypeStruct((NC, RPT, L), jnp.float32)]
        scratch += [
            pltpu.VMEM((RPT, L), jnp.float32),
            pltpu.VMEM((CPT, CH), jnp.int32),
        ]
    scratch.append(pltpu.VMEM_SHARED((NP, width), jnp.float32))
    if with_cnt:
        scratch.append(pltpu.VMEM_SHARED((RPT, L), jnp.float32))
    scratch += [pltpu.SemaphoreType.DMA] * 4

    def body(*refs):
        return _seg_body(with_cnt, *refs)

    return pl.kernel(
        body,
        out_type=out_type,
        mesh=_MESH,
        scratch_types=scratch,
        compiler_params=_SC_PARAMS,
    )


_seg64c = _make_seg(True)
_seg64 = _make_seg(False)


def _seg1_body(ei_hbm, p_hbm, z_hbm, ii_hbm, out_hbm,
               eidx, pv, acc_v, iidx_v, red_sh):
    c = lax.axis_index("c")
    s = lax.axis_index("s")
    wid = s * NC + c
    row0 = wid * CPW

    pltpu.sync_copy(z_hbm, acc_v)
    @pl.when(s == 0)
    def _():
        pltpu.sync_copy(acc_v, red_sh)
    pltpu.sync_copy(ii_hbm, iidx_v)
    pltpu.sync_copy(ei_hbm.at[pl.ds(row0, CPW)], eidx.at[pl.ds(0, CPW)])
    @pl.when(wid < 4)
    def _():
        pltpu.sync_copy(ei_hbm.at[pl.ds(NW * CPW + wid, 1)],
                        eidx.at[pl.ds(CPW, 1)])
    pltpu.sync_copy(p_hbm, pv)
    plsc.subcore_barrier()

    def do_chunk(i):
        for u in range(CH // L):
            sv = eidx[i, 0, pl.ds(u * L, L)]
            d = eidx[i, 1, pl.ds(u * L, L)]
            v = plsc.load_gather(pv, [sv])
            plsc.addupdate_scatter(
                acc_v, [jnp.right_shift(d, 4), jnp.bitwise_and(d, 15)], v
            )

    @pl.loop(0, CPW)
    def _(i):
        do_chunk(i)

    @pl.when(wid < 4)
    def _():
        do_chunk(CPW)

    for k in range(CPT):
        pltpu.sync_copy(acc_v.at[pl.ds(k * CH, CH), :],
                        red_sh.at[iidx_v.at[k]], add=True)
    plsc.subcore_barrier()

    @pl.when(s == 0)
    def _():
        pltpu.sync_copy(red_sh, acc_v)
        pltpu.sync_copy(acc_v, out_hbm.at[c])


_seg1 = pl.kernel(
    _seg1_body,
    out_type=jax.ShapeDtypeStruct((NC, RPT, L), jnp.float32),
    mesh=_MESH,
    scratch_types=[
        pltpu.VMEM((CPW + 1, 2, CH), jnp.int32),
        pltpu.VMEM((NP,), jnp.float32),
        pltpu.VMEM((RPT, L), jnp.float32),
        pltpu.VMEM((CPT, CH), jnp.int32),
        pltpu.VMEM_SHARED((RPT, L), jnp.float32),
    ],
    compiler_params=_SC_PARAMS,
)


BR = 2000


def _pre_body(h_ref, wl_ref, bl_ref, wr_ref, p_ref, r_ref):
    h = h_ref[...]
    p_ref[...] = jnp.dot(h, wl_ref[...], preferred_element_type=jnp.float32)
    r_ref[...] = (
        jnp.dot(h, wr_ref[...], preferred_element_type=jnp.float32) + bl_ref[...]
    )


def _comb1_body(a_ref, c0_ref, c1_ref, rp_ref, wl_ref, bl_ref, wr_ref,
                p_ref, r_ref, inv_ref):
    cnt = c0_ref[...] + c1_ref[...]
    inv = 1.0 / jnp.maximum(cnt, 1.0)
    h = jnp.maximum((a_ref[0] + a_ref[1]) * inv + rp_ref[...], 0.0)
    p_ref[...] = jnp.dot(h, wl_ref[...], preferred_element_type=jnp.float32)
    r_ref[...] = (
        jnp.dot(h, wr_ref[...], preferred_element_type=jnp.float32) + bl_ref[...]
    )
    inv_ref[...] = inv


def _comb_body(a_ref, inv_ref, rp_ref, wl_ref, bl_ref, wr_ref, p_ref, r_ref):
    h = jnp.maximum(
        (a_ref[0] + a_ref[1]) * inv_ref[...] + rp_ref[...], 0.0
    )
    p_ref[...] = jnp.dot(h, wl_ref[...], preferred_element_type=jnp.float32)
    r_ref[...] = (
        jnp.dot(h, wr_ref[...], preferred_element_type=jnp.float32) + bl_ref[...]
    )


def _final_body(a0_ref, a1_ref, inv_ref, r3_ref, o_ref):
    o_ref[...] = (a0_ref[...] + a1_ref[...]) * inv_ref[...] + r3_ref[...]


def _pre(x, wl, bl, wr):
    din, dout = wl.shape
    return pl.pallas_call(
        _pre_body,
        grid=(N // BR,),
        in_specs=[
            pl.BlockSpec((BR, din), lambda i: (i, 0)),
            pl.BlockSpec((din, dout), lambda i: (0, 0)),
            pl.BlockSpec((1, dout), lambda i: (0, 0)),
            pl.BlockSpec((din, dout), lambda i: (0, 0)),
        ],
        out_specs=[pl.BlockSpec((BR, dout), lambda i: (i, 0))] * 2,
        out_shape=[jax.ShapeDtypeStruct((N, dout), jnp.float32)] * 2,
    )(x, wl, bl, wr)


def _comb1(a, c0, c1, rp, wl, bl, wr):
    din, dout = wl.shape
    return pl.pallas_call(
        _comb1_body,
        grid=(N // BR,),
        in_specs=[
            pl.BlockSpec((NC, BR, 64), lambda i: (0, i, 0)),
            pl.BlockSpec((BR, 1), lambda i: (i, 0)),
            pl.BlockSpec((BR, 1), lambda i: (i, 0)),
            pl.BlockSpec((BR, din), lambda i: (i, 0)),
            pl.BlockSpec((din, dout), lambda i: (0, 0)),
            pl.BlockSpec((1, dout), lambda i: (0, 0)),
            pl.BlockSpec((din, dout), lambda i: (0, 0)),
        ],
        out_specs=[
            pl.BlockSpec((BR, dout), lambda i: (i, 0)),
            pl.BlockSpec((BR, dout), lambda i: (i, 0)),
            pl.BlockSpec((BR, 1), lambda i: (i, 0)),
        ],
        out_shape=[
            jax.ShapeDtypeStruct((N, dout), jnp.float32),
            jax.ShapeDtypeStruct((N, dout), jnp.float32),
            jax.ShapeDtypeStruct((N, 1), jnp.float32),
        ],
    )(a, c0, c1, rp, wl, bl, wr)


def _comb(a, inv, rp, wl, bl, wr):
    din, dout = wl.shape
    return pl.pallas_call(
        _comb_body,
        grid=(N // BR,),
        in_specs=[
            pl.BlockSpec((NC, BR, 64), lambda i: (0, i, 0)),
            pl.BlockSpec((BR, 1), lambda i: (i, 0)),
            pl.BlockSpec((BR, din), lambda i: (i, 0)),
            pl.BlockSpec((din, dout), lambda i: (0, 0)),
            pl.BlockSpec((1, dout), lambda i: (0, 0)),
            pl.BlockSpec((din, dout), lambda i: (0, 0)),
        ],
        out_specs=[pl.BlockSpec((BR, dout), lambda i: (i, 0))] * 2,
        out_shape=[jax.ShapeDtypeStruct((N, dout), jnp.float32)] * 2,
    )(a, inv, rp, wl, bl, wr)


def _final(a30, a31, inv, r3):
    spec = pl.BlockSpec((N, 1), lambda: (0, 0))
    return pl.pallas_call(
        _final_body,
        in_specs=[spec] * 4,
        out_specs=spec,
        out_shape=jax.ShapeDtypeStruct((N, 1), jnp.float32),
    )(a30, a31, inv, r3)


def kernel(x, edge_index, Wl0, bl0, Wr0, Wl1, bl1, Wr1, Wl2, bl2, Wr2,
           Wl3, bl3, Wr3):
    ei3 = edge_index.reshape(2, NCHUNK, CH).transpose(1, 0, 2)
    z64 = jnp.zeros((CH, 64), jnp.float32)
    z16 = jnp.zeros((RPT, L), jnp.float32)
    iid = jnp.arange(RPT, dtype=jnp.int32).reshape(CPT, CH)

    p, r = _pre(x, Wl0, bl0.reshape(1, -1), Wr0)
    a, ac = _seg64c(ei3, p, z64, z16, iid)
    acr = ac.reshape(NC, NP)
    c0 = acr[0, :N].reshape(N, 1)
    c1 = acr[1, :N].reshape(N, 1)
    p, r, inv = _comb1(a, c0, c1, r, Wl1, bl1.reshape(1, -1), Wr1)
    for wl, bl, wr in ((Wl2, bl2, Wr2), (Wl3, bl3, Wr3)):
        a = _seg64(ei3, p, z64)
        p, r = _comb(a, inv, r, wl, bl.reshape(1, -1), wr)

    p3 = jnp.pad(p.reshape(-1), (0, NP - N))
    a3 = _seg1(ei3, p3, z16, iid)
    a3r = a3.reshape(NC, NP)
    out = _final(a3r[0, :N].reshape(N, 1), a3r[1, :N].reshape(N, 1), inv, r)
    return out.reshape(-1)


# speedup vs baseline: 19.1185x; 1.1041x over previous
import jax
import jax.numpy as jnp
from jax import lax
from jax.experimental import pallas as pl
from jax.experimental.pallas import tpu as pltpu
from jax.experimental.pallas import tpu_sc as plsc

N = 10000
E = 320000
NP = 10240
NC, NS, L = 2, 16, 16
NW = NC * NS
CH = 128
NCHUNK = E // CH
EPW = E // NW
RPT = NP // NS
CPW = NCHUNK // NW
K = 3
NWIN = CPW // K
NPAIR = NWIN // 2
CPT = RPT // CH
HN = NP // 2
NH = N // 2

_MESH = plsc.VectorSubcoreMesh(
    core_axis_name="c", subcore_axis_name="s", num_cores=NC, num_subcores=NS
)
_SC_PARAMS = pltpu.CompilerParams(
    needs_layout_passes=False, use_tc_tiling_on_sc=False
)


def _seg_body(with_cnt, *refs):
    if with_cnt:
        (ei_hbm, p_hbm, z_hbm, z16_hbm, ii_hbm, out_hbm, outc_hbm,
         eidx, ra, rb, cacc, iidx_v, acc_sh, red_sh, ga, gb, sa, sb) = refs
    else:
        (ei_hbm, p_hbm, z_hbm, out_hbm,
         eidx, ra, rb, acc_sh, ga, gb, sa, sb) = refs
    c = lax.axis_index("c")
    s = lax.axis_index("s")
    wid = s * NC + c
    row0 = wid * CPW

    pltpu.sync_copy(ei_hbm.at[pl.ds(row0, CPW)], eidx.at[pl.ds(0, CPW)])
    @pl.when(wid < 4)
    def _():
        xrow = NW * CPW + wid
        pltpu.sync_copy(ei_hbm.at[pl.ds(xrow, 1)], eidx.at[pl.ds(CPW, 1)])

    pltpu.sync_copy(z_hbm, ra.at[0])
    for k in range(CPT):
        pltpu.async_copy(ra.at[0], acc_sh.at[pl.ds(s * RPT + k * CH, CH), :], ga)
    for k in range(CPT):
        pltpu.make_async_copy(z_hbm, ra.at[0], ga).wait()
    if with_cnt:
        pltpu.sync_copy(z16_hbm, cacc)
        pltpu.sync_copy(ii_hbm, iidx_v)

        @pl.when(s == 0)
        def _():
            pltpu.sync_copy(cacc, red_sh)
    plsc.subcore_barrier()

    ones = jnp.ones((L,), jnp.float32)

    def _split_idx(d):
        idx = jnp.right_shift(d, 1) + jnp.bitwise_and(d, 1) * HN
        return [jnp.right_shift(idx, 4), jnp.bitwise_and(idx, 15)]

    def fire(w, buf, sem):
        for t in range(K):
            pltpu.async_copy(p_hbm.at[eidx.at[w * K + t, 0]], buf.at[t], sem)

    def drain(w, buf, sem):
        for t in range(K):
            pltpu.make_async_copy(
                p_hbm.at[eidx.at[w * K + t, 0]], buf.at[t], sem).wait()

    def scat(w, buf, sem):
        for t in range(K):
            pltpu.async_copy(buf.at[t], acc_sh.at[eidx.at[w * K + t, 1]],
                             sem, add=True)

    def scat_drain(w, buf, sem):
        for t in range(K):
            pltpu.make_async_copy(buf.at[t], acc_sh.at[eidx.at[w * K + t, 1]],
                                  sem).wait()

    def count(w):
        if not with_cnt:
            return
        for t in range(K):
            for u in range(CH // L):
                d = eidx[w * K + t, 1, pl.ds(u * L, L)]
                plsc.addupdate_scatter(cacc, _split_idx(d), ones)

    fire(0, ra, ga)

    @pl.loop(0, NPAIR)
    def _(j):
        wa = 2 * j

        @pl.when(j > 0)
        def _():
            scat_drain(wa - 1, rb, sb)

        fire(wa + 1, rb, gb)
        count(wa)
        drain(wa, ra, ga)
        scat(wa, ra, sa)
        count(wa + 1)
        drain(wa + 1, rb, gb)
        scat_drain(wa, ra, sa)

        @pl.when(j < NPAIR - 1)
        def _():
            fire(wa + 2, ra, ga)

        scat(wa + 1, rb, sb)

    scat_drain(2 * NPAIR - 1, rb, sb)

    @pl.when(wid < 4)
    def _():
        pltpu.async_copy(p_hbm.at[eidx.at[CPW, 0]], ra.at[0], ga).wait()
        pltpu.sync_copy(ra.at[0], acc_sh.at[eidx.at[CPW, 1]], add=True)
        if with_cnt:
            for u in range(CH // L):
                d = eidx[CPW, 1, pl.ds(u * L, L)]
                plsc.addupdate_scatter(cacc, _split_idx(d), ones)

    if with_cnt:
        for k in range(CPT):
            pltpu.sync_copy(cacc.at[pl.ds(k * CH, CH), :],
                            red_sh.at[iidx_v.at[k]], add=True)

    plsc.subcore_barrier()
    bufs = [ra.at[0], ra.at[1], ra.at[2], rb.at[0], rb.at[1]]
    for k in range(CPT):
        pltpu.async_copy(acc_sh.at[pl.ds(s * RPT + k * CH, CH), :], bufs[k], ga)
    for k in range(CPT):
        pltpu.make_async_copy(acc_sh.at[pl.ds(s * RPT + k * CH, CH), :],
                              bufs[k], ga).wait()
    for k in range(CPT):
        pltpu.async_copy(bufs[k], out_hbm.at[c, pl.ds(s * RPT + k * CH, CH), :], ga)
    for k in range(CPT):
        pltpu.make_async_copy(bufs[k],
                              out_hbm.at[c, pl.ds(s * RPT + k * CH, CH), :],
                              ga).wait()
    if with_cnt:
        @pl.when(s == 0)
        def _():
            pltpu.sync_copy(red_sh, cacc)
            pltpu.sync_copy(cacc, outc_hbm.at[c])


def _make_seg(with_cnt):
    width = 64
    out_type = jax.ShapeDtypeStruct((NC, NP, width), jnp.float32)
    scratch = [
        pltpu.VMEM((CPW + 1, 2, CH), jnp.int32),
        pltpu.VMEM((K, CH, width), jnp.float32),
        pltpu.VMEM((K, CH, width), jnp.float32),
    ]
    if with_cnt:
        out_type = [out_type, jax.ShapeDtypeStruct((NC, RPT, L), jnp.float32)]
        scratch += [
            pltpu.VMEM((RPT, L), jnp.float32),
            pltpu.VMEM((CPT, CH), jnp.int32),
        ]
    scratch.append(pltpu.VMEM_SHARED((NP, width), jnp.float32))
    if with_cnt:
        scratch.append(pltpu.VMEM_SHARED((RPT, L), jnp.float32))
    scratch += [pltpu.SemaphoreType.DMA] * 4

    def body(*refs):
        return _seg_body(with_cnt, *refs)

    return pl.kernel(
        body,
        out_type=out_type,
        mesh=_MESH,
        scratch_types=scratch,
        compiler_params=_SC_PARAMS,
    )


_seg64c = _make_seg(True)
_seg64 = _make_seg(False)


def _seg1_body(ei_hbm, p_hbm, z_hbm, ii_hbm, out_hbm,
               eidx, pv, acc_v, iidx_v, red_sh):
    c = lax.axis_index("c")
    s = lax.axis_index("s")
    wid = s * NC + c
    row0 = wid * CPW

    pltpu.sync_copy(z_hbm, acc_v)
    @pl.when(s == 0)
    def _():
        pltpu.sync_copy(acc_v, red_sh)
    pltpu.sync_copy(ii_hbm, iidx_v)
    pltpu.sync_copy(ei_hbm.at[pl.ds(row0, CPW)], eidx.at[pl.ds(0, CPW)])
    @pl.when(wid < 4)
    def _():
        pltpu.sync_copy(ei_hbm.at[pl.ds(NW * CPW + wid, 1)],
                        eidx.at[pl.ds(CPW, 1)])
    pltpu.sync_copy(p_hbm, pv)
    plsc.subcore_barrier()

    def do_chunk(i):
        for u in range(CH // L):
            sv = eidx[i, 0, pl.ds(u * L, L)]
            d = eidx[i, 1, pl.ds(u * L, L)]
            v = plsc.load_gather(pv, [sv])
            idx = jnp.right_shift(d, 1) + jnp.bitwise_and(d, 1) * HN
            plsc.addupdate_scatter(
                acc_v, [jnp.right_shift(idx, 4), jnp.bitwise_and(idx, 15)], v
            )

    @pl.loop(0, CPW)
    def _(i):
        do_chunk(i)

    @pl.when(wid < 4)
    def _():
        do_chunk(CPW)

    for k in range(CPT):
        pltpu.sync_copy(acc_v.at[pl.ds(k * CH, CH), :],
                        red_sh.at[iidx_v.at[k]], add=True)
    plsc.subcore_barrier()

    @pl.when(s == 0)
    def _():
        pltpu.sync_copy(red_sh, acc_v)
        pltpu.sync_copy(acc_v, out_hbm.at[c])


_seg1 = pl.kernel(
    _seg1_body,
    out_type=jax.ShapeDtypeStruct((NC, RPT, L), jnp.float32),
    mesh=_MESH,
    scratch_types=[
        pltpu.VMEM((CPW + 1, 2, CH), jnp.int32),
        pltpu.VMEM((NP,), jnp.float32),
        pltpu.VMEM((RPT, L), jnp.float32),
        pltpu.VMEM((CPT, CH), jnp.int32),
        pltpu.VMEM_SHARED((RPT, L), jnp.float32),
    ],
    compiler_params=_SC_PARAMS,
)


BR = 2000


def _pre_body(h_ref, wl_ref, bl_ref, wr_ref, p_ref, r_ref):
    h = h_ref[...]
    p_ref[...] = jnp.dot(h, wl_ref[...], preferred_element_type=jnp.float32)
    r_ref[...] = (
        jnp.dot(h, wr_ref[...], preferred_element_type=jnp.float32) + bl_ref[...]
    )


def _comb1_body(a_ref, c0e, c0o, c1e, c1o, re_ref, ro_ref,
                wl_ref, bl_ref, wr_ref,
                p_ref, rne_ref, rno_ref, ive_ref, ivo_ref):
    s = a_ref[0] + a_ref[1]
    inve = 1.0 / jnp.maximum(c0e[...] + c1e[...], 1.0)
    invo = 1.0 / jnp.maximum(c0o[...] + c1o[...], 1.0)
    he = jnp.maximum(s[:, :64] * inve + re_ref[...], 0.0)
    ho = jnp.maximum(s[:, 64:] * invo + ro_ref[...], 0.0)
    wl = wl_ref[...]
    wr = wr_ref[...]
    p_ref[...] = jnp.concatenate(
        [jnp.dot(he, wl, preferred_element_type=jnp.float32),
         jnp.dot(ho, wl, preferred_element_type=jnp.float32)], axis=1)
    rne_ref[...] = jnp.dot(he, wr, preferred_element_type=jnp.float32) + bl_ref[...]
    rno_ref[...] = jnp.dot(ho, wr, preferred_element_type=jnp.float32) + bl_ref[...]
    ive_ref[...] = inve
    ivo_ref[...] = invo


def _comb_body(a_ref, ive, ivo, re_ref, ro_ref, wl_ref, bl_ref, wr_ref,
               p_ref, rne_ref, rno_ref):
    s = a_ref[0] + a_ref[1]
    he = jnp.maximum(s[:, :64] * ive[...] + re_ref[...], 0.0)
    ho = jnp.maximum(s[:, 64:] * ivo[...] + ro_ref[...], 0.0)
    wl = wl_ref[...]
    wr = wr_ref[...]
    p_ref[...] = jnp.concatenate(
        [jnp.dot(he, wl, preferred_element_type=jnp.float32),
         jnp.dot(ho, wl, preferred_element_type=jnp.float32)], axis=1)
    rne_ref[...] = jnp.dot(he, wr, preferred_element_type=jnp.float32) + bl_ref[...]
    rno_ref[...] = jnp.dot(ho, wr, preferred_element_type=jnp.float32) + bl_ref[...]


def _final_body(ae0, ao0, ae1, ao1, ive, ivo, re_ref, ro_ref, o_ref):
    oe = (ae0[...] + ae1[...]) * ive[...] + re_ref[...]
    oo = (ao0[...] + ao1[...]) * ivo[...] + ro_ref[...]
    o_ref[...] = jnp.concatenate([oe, oo], axis=1)


def _pre(x, wl, bl, wr):
    din, dout = wl.shape
    return pl.pallas_call(
        _pre_body,
        grid=(N // BR,),
        in_specs=[
            pl.BlockSpec((BR, din), lambda i: (i, 0)),
            pl.BlockSpec((din, dout), lambda i: (0, 0)),
            pl.BlockSpec((1, dout), lambda i: (0, 0)),
            pl.BlockSpec((din, dout), lambda i: (0, 0)),
        ],
        out_specs=[pl.BlockSpec((BR, dout), lambda i: (i, 0))] * 2,
        out_shape=[jax.ShapeDtypeStruct((N, dout), jnp.float32)] * 2,
    )(x, wl, bl, wr)


BRH = 1000


def _comb1(a128, c0e, c0o, c1e, c1o, re, ro, wl, bl, wr):
    din, dout = wl.shape
    rspec = pl.BlockSpec((BRH, din), lambda i: (i, 0))
    cspec = pl.BlockSpec((BRH, 1), lambda i: (i, 0))
    ospec = pl.BlockSpec((BRH, dout), lambda i: (i, 0))
    return pl.pallas_call(
        _comb1_body,
        grid=(NH // BRH,),
        in_specs=[
            pl.BlockSpec((NC, BRH, 128), lambda i: (0, i, 0)),
            cspec, cspec, cspec, cspec, rspec, rspec,
            pl.BlockSpec((din, dout), lambda i: (0, 0)),
            pl.BlockSpec((1, dout), lambda i: (0, 0)),
            pl.BlockSpec((din, dout), lambda i: (0, 0)),
        ],
        out_specs=[
            pl.BlockSpec((BRH, 2 * dout), lambda i: (i, 0)),
            ospec, ospec, cspec, cspec,
        ],
        out_shape=[
            jax.ShapeDtypeStruct((NH, 2 * dout), jnp.float32),
            jax.ShapeDtypeStruct((NH, dout), jnp.float32),
            jax.ShapeDtypeStruct((NH, dout), jnp.float32),
            jax.ShapeDtypeStruct((NH, 1), jnp.float32),
            jax.ShapeDtypeStruct((NH, 1), jnp.float32),
        ],
    )(a128, c0e, c0o, c1e, c1o, re, ro, wl, bl, wr)


def _comb(a128, ive, ivo, re, ro, wl, bl, wr):
    din, dout = wl.shape
    rspec = pl.BlockSpec((BRH, din), lambda i: (i, 0))
    cspec = pl.BlockSpec((BRH, 1), lambda i: (i, 0))
    ospec = pl.BlockSpec((BRH, dout), lambda i: (i, 0))
    return pl.pallas_call(
        _comb_body,
        grid=(NH // BRH,),
        in_specs=[
            pl.BlockSpec((NC, BRH, 128), lambda i: (0, i, 0)),
            cspec, cspec, rspec, rspec,
            pl.BlockSpec((din, dout), lambda i: (0, 0)),
            pl.BlockSpec((1, dout), lambda i: (0, 0)),
            pl.BlockSpec((din, dout), lambda i: (0, 0)),
        ],
        out_specs=[
            pl.BlockSpec((BRH, 2 * dout), lambda i: (i, 0)),
            ospec, ospec,
        ],
        out_shape=[
            jax.ShapeDtypeStruct((NH, 2 * dout), jnp.float32),
            jax.ShapeDtypeStruct((NH, dout), jnp.float32),
            jax.ShapeDtypeStruct((NH, dout), jnp.float32),
        ],
    )(a128, ive, ivo, re, ro, wl, bl, wr)


def _final(ae0, ao0, ae1, ao1, ive, ivo, re, ro):
    spec = pl.BlockSpec((NH, 1), lambda: (0, 0))
    return pl.pallas_call(
        _final_body,
        in_specs=[spec] * 8,
        out_specs=pl.BlockSpec((NH, 2), lambda: (0, 0)),
        out_shape=jax.ShapeDtypeStruct((NH, 2), jnp.float32),
    )(ae0, ao0, ae1, ao1, ive, ivo, re, ro)


def kernel(x, edge_index, Wl0, bl0, Wr0, Wl1, bl1, Wr1, Wl2, bl2, Wr2,
           Wl3, bl3, Wr3):
    ei3 = edge_index.reshape(2, NCHUNK, CH).transpose(1, 0, 2)
    z64 = jnp.zeros((CH, 64), jnp.float32)
    z16 = jnp.zeros((RPT, L), jnp.float32)
    iid = jnp.arange(RPT, dtype=jnp.int32).reshape(CPT, CH)

    p, r = _pre(x, Wl0, bl0.reshape(1, -1), Wr0)
    re = r[0::2]
    ro = r[1::2]
    a, ac = _seg64c(ei3, p, z64, z16, iid)
    acl = ac.reshape(NC, NP)
    c0e = acl[0, :NH].reshape(NH, 1)
    c0o = acl[0, HN:HN + NH].reshape(NH, 1)
    c1e = acl[1, :NH].reshape(NH, 1)
    c1o = acl[1, HN:HN + NH].reshape(NH, 1)
    p, re, ro, ive, ivo = _comb1(
        a.reshape(NC, HN, 128), c0e, c0o, c1e, c1o, re, ro,
        Wl1, bl1.reshape(1, -1), Wr1)
    for wl, bl, wr in ((Wl2, bl2, Wr2), (Wl3, bl3, Wr3)):
        a = _seg64(ei3, p.reshape(N, 64), z64)
        p, re, ro = _comb(a.reshape(NC, HN, 128), ive, ivo, re, ro,
                          wl, bl.reshape(1, -1), wr)

    p3 = jnp.pad(p.reshape(-1), (0, NP - N))
    a3 = _seg1(ei3, p3, z16, iid)
    a3l = a3.reshape(NC, NP)
    out = _final(a3l[0, :NH].reshape(NH, 1), a3l[0, HN:HN + NH].reshape(NH, 1),
                 a3l[1, :NH].reshape(NH, 1), a3l[1, HN:HN + NH].reshape(NH, 1),
                 ive, ivo, re, ro)
    return out.reshape(-1)


# speedup vs baseline: 19.2894x; 1.0089x over previous
import jax
import jax.numpy as jnp
from jax import lax
from jax.experimental import pallas as pl
from jax.experimental.pallas import tpu as pltpu
from jax.experimental.pallas import tpu_sc as plsc

N = 10000
E = 320000
NP = 10240
NC, NS, L = 2, 16, 16
NW = NC * NS
CH = 128
NCHUNK = E // CH
EPW = E // NW
RPT = NP // NS
CPW = NCHUNK // NW
K = 3
NWIN = CPW // K
NPAIR = NWIN // 2
CPT = RPT // CH
HN = NP // 2
NH = N // 2

_MESH = plsc.VectorSubcoreMesh(
    core_axis_name="c", subcore_axis_name="s", num_cores=NC, num_subcores=NS
)
_SC_PARAMS = pltpu.CompilerParams(
    needs_layout_passes=False, use_tc_tiling_on_sc=False
)


def _seg_body(with_cnt, *refs):
    if with_cnt:
        (ei_hbm, p_hbm, z_hbm, z16_hbm, ii_hbm, out_hbm, outc_hbm,
         eidx, ra, rb, cacc, iidx_v, acc_sh, red_sh, ga, gb, sa, sb) = refs
    else:
        (ei_hbm, p_hbm, z_hbm, out_hbm,
         eidx, ra, rb, acc_sh, ga, gb, sa, sb) = refs
    c = lax.axis_index("c")
    s = lax.axis_index("s")
    wid = s * NC + c
    row0 = wid * CPW

    pltpu.sync_copy(ei_hbm.at[pl.ds(row0, CPW)], eidx.at[pl.ds(0, CPW)])
    @pl.when(wid < 4)
    def _():
        xrow = NW * CPW + wid
        pltpu.sync_copy(ei_hbm.at[pl.ds(xrow, 1)], eidx.at[pl.ds(CPW, 1)])

    pltpu.sync_copy(z_hbm, ra.at[0])
    for k in range(CPT):
        pltpu.async_copy(ra.at[0], acc_sh.at[pl.ds(s * RPT + k * CH, CH), :], ga)
    for k in range(CPT):
        pltpu.make_async_copy(z_hbm, ra.at[0], ga).wait()
    if with_cnt:
        pltpu.sync_copy(z16_hbm, cacc)
        pltpu.sync_copy(ii_hbm, iidx_v)

        @pl.when(s == 0)
        def _():
            pltpu.sync_copy(cacc, red_sh)
    plsc.subcore_barrier()

    ones = jnp.ones((L,), jnp.float32)

    def _split_idx(d):
        idx = jnp.right_shift(d, 1) + jnp.bitwise_and(d, 1) * HN
        return [jnp.right_shift(idx, 4), jnp.bitwise_and(idx, 15)]

    def fire(w, buf, sem):
        for t in range(K):
            pltpu.async_copy(p_hbm.at[eidx.at[w * K + t, 0]], buf.at[t], sem)

    def drain(w, buf, sem):
        for t in range(K):
            pltpu.make_async_copy(
                p_hbm.at[eidx.at[w * K + t, 0]], buf.at[t], sem).wait()

    def scat(w, buf, sem):
        for t in range(K):
            pltpu.async_copy(buf.at[t], acc_sh.at[eidx.at[w * K + t, 1]],
                             sem, add=True)

    def scat_drain(w, buf, sem):
        for t in range(K):
            pltpu.make_async_copy(buf.at[t], acc_sh.at[eidx.at[w * K + t, 1]],
                                  sem).wait()

    def count(w):
        if not with_cnt:
            return
        for t in range(K):
            for u in range(CH // L):
                d = eidx[w * K + t, 1, pl.ds(u * L, L)]
                plsc.addupdate_scatter(cacc, _split_idx(d), ones)

    fire(0, ra, ga)

    @pl.loop(0, NPAIR)
    def _(j):
        wa = 2 * j

        @pl.when(j > 0)
        def _():
            scat_drain(wa - 1, rb, sb)

        fire(wa + 1, rb, gb)
        count(wa)
        drain(wa, ra, ga)
        scat(wa, ra, sa)
        count(wa + 1)
        drain(wa + 1, rb, gb)
        scat_drain(wa, ra, sa)

        @pl.when(j < NPAIR - 1)
        def _():
            fire(wa + 2, ra, ga)

        scat(wa + 1, rb, sb)

    scat_drain(2 * NPAIR - 1, rb, sb)

    @pl.when(wid < 4)
    def _():
        pltpu.async_copy(p_hbm.at[eidx.at[CPW, 0]], ra.at[0], ga).wait()
        pltpu.sync_copy(ra.at[0], acc_sh.at[eidx.at[CPW, 1]], add=True)
        if with_cnt:
            for u in range(CH // L):
                d = eidx[CPW, 1, pl.ds(u * L, L)]
                plsc.addupdate_scatter(cacc, _split_idx(d), ones)

    if with_cnt:
        for k in range(CPT):
            pltpu.sync_copy(cacc.at[pl.ds(k * CH, CH), :],
                            red_sh.at[iidx_v.at[k]], add=True)

    plsc.subcore_barrier()
    bufs = [ra.at[0], ra.at[1], ra.at[2], rb.at[0], rb.at[1]]
    for k in range(CPT):
        pltpu.async_copy(acc_sh.at[pl.ds(s * RPT + k * CH, CH), :], bufs[k], ga)
    for k in range(CPT):
        pltpu.make_async_copy(acc_sh.at[pl.ds(s * RPT + k * CH, CH), :],
                              bufs[k], ga).wait()
    for k in range(CPT):
        pltpu.async_copy(bufs[k], out_hbm.at[c, pl.ds(s * RPT + k * CH, CH), :], ga)
    for k in range(CPT):
        pltpu.make_async_copy(bufs[k],
                              out_hbm.at[c, pl.ds(s * RPT + k * CH, CH), :],
                              ga).wait()
    if with_cnt:
        @pl.when(s == 0)
        def _():
            pltpu.sync_copy(red_sh, cacc)
            pltpu.sync_copy(cacc, outc_hbm.at[c])


def _make_seg(with_cnt):
    width = 64
    out_type = jax.ShapeDtypeStruct((NC, NP, width), jnp.float32)
    scratch = [
        pltpu.VMEM((CPW + 1, 2, CH), jnp.int32),
        pltpu.VMEM((K, CH, width), jnp.float32),
        pltpu.VMEM((K, CH, width), jnp.float32),
    ]
    if with_cnt:
        out_type = [out_type, jax.ShapeDtypeStruct((NC, RPT, L), jnp.float32)]
        scratch += [
            pltpu.VMEM((RPT, L), jnp.float32),
            pltpu.VMEM((CPT, CH), jnp.int32),
        ]
    scratch.append(pltpu.VMEM_SHARED((NP, width), jnp.float32))
    if with_cnt:
        scratch.append(pltpu.VMEM_SHARED((RPT, L), jnp.float32))
    scratch += [pltpu.SemaphoreType.DMA] * 4

    def body(*refs):
        return _seg_body(with_cnt, *refs)

    return pl.kernel(
        body,
        out_type=out_type,
        mesh=_MESH,
        scratch_types=scratch,
        compiler_params=_SC_PARAMS,
    )


_seg64c = _make_seg(True)
_seg64 = _make_seg(False)


def _seg1_body(ei_hbm, p_hbm, z_hbm, ii_hbm, out_hbm,
               eidx, pv, acc_v, iidx_v, red_sh):
    c = lax.axis_index("c")
    s = lax.axis_index("s")
    wid = s * NC + c
    row0 = wid * CPW

    pltpu.sync_copy(z_hbm, acc_v)
    @pl.when(s == 0)
    def _():
        pltpu.sync_copy(acc_v, red_sh)
    pltpu.sync_copy(ii_hbm, iidx_v)
    pltpu.sync_copy(ei_hbm.at[pl.ds(row0, CPW)], eidx.at[pl.ds(0, CPW)])
    @pl.when(wid < 4)
    def _():
        pltpu.sync_copy(ei_hbm.at[pl.ds(NW * CPW + wid, 1)],
                        eidx.at[pl.ds(CPW, 1)])
    pltpu.sync_copy(p_hbm, pv)
    plsc.subcore_barrier()

    def do_chunk(i):
        for u in range(CH // L):
            sv = eidx[i, 0, pl.ds(u * L, L)]
            d = eidx[i, 1, pl.ds(u * L, L)]
            v = plsc.load_gather(pv, [sv])
            idx = jnp.right_shift(d, 1) + jnp.bitwise_and(d, 1) * HN
            plsc.addupdate_scatter(
                acc_v, [jnp.right_shift(idx, 4), jnp.bitwise_and(idx, 15)], v
            )

    @pl.loop(0, CPW)
    def _(i):
        do_chunk(i)

    @pl.when(wid < 4)
    def _():
        do_chunk(CPW)

    for k in range(CPT):
        pltpu.sync_copy(acc_v.at[pl.ds(k * CH, CH), :],
                        red_sh.at[iidx_v.at[k]], add=True)
    plsc.subcore_barrier()

    @pl.when(s == 0)
    def _():
        pltpu.sync_copy(red_sh, acc_v)
        pltpu.sync_copy(acc_v, out_hbm.at[c])


_seg1 = pl.kernel(
    _seg1_body,
    out_type=jax.ShapeDtypeStruct((NC, RPT, L), jnp.float32),
    mesh=_MESH,
    scratch_types=[
        pltpu.VMEM((CPW + 1, 2, CH), jnp.int32),
        pltpu.VMEM((NP,), jnp.float32),
        pltpu.VMEM((RPT, L), jnp.float32),
        pltpu.VMEM((CPT, CH), jnp.int32),
        pltpu.VMEM_SHARED((RPT, L), jnp.float32),
    ],
    compiler_params=_SC_PARAMS,
)


BR = 2000


def _pre_body(h_ref, wl_ref, bl_ref, wr_ref, p_ref, r_ref):
    h = h_ref[...]
    p_ref[...] = jnp.dot(h, wl_ref[...], preferred_element_type=jnp.float32)
    r_ref[...] = (
        jnp.dot(h, wr_ref[...], preferred_element_type=jnp.float32) + bl_ref[...]
    )


def _comb1_body(a_ref, c0e, c0o, c1e, c1o, rp_ref,
                wl_ref, bl_ref, wr_ref,
                p_ref, rne_ref, rno_ref, ive_ref, ivo_ref):
    s = a_ref[0] + a_ref[1]
    rp = rp_ref[...]
    inve = 1.0 / jnp.maximum(c0e[...] + c1e[...], 1.0)
    invo = 1.0 / jnp.maximum(c0o[...] + c1o[...], 1.0)
    he = jnp.maximum(s[:, :64] * inve + rp[:, :64], 0.0)
    ho = jnp.maximum(s[:, 64:] * invo + rp[:, 64:], 0.0)
    wl = wl_ref[...]
    wr = wr_ref[...]
    p_ref[...] = jnp.concatenate(
        [jnp.dot(he, wl, preferred_element_type=jnp.float32),
         jnp.dot(ho, wl, preferred_element_type=jnp.float32)], axis=1)
    rne_ref[...] = jnp.dot(he, wr, preferred_element_type=jnp.float32) + bl_ref[...]
    rno_ref[...] = jnp.dot(ho, wr, preferred_element_type=jnp.float32) + bl_ref[...]
    ive_ref[...] = inve
    ivo_ref[...] = invo


def _comb_body(a_ref, ive, ivo, re_ref, ro_ref, wl_ref, bl_ref, wr_ref,
               p_ref, rne_ref, rno_ref):
    s = a_ref[0] + a_ref[1]
    he = jnp.maximum(s[:, :64] * ive[...] + re_ref[...], 0.0)
    ho = jnp.maximum(s[:, 64:] * ivo[...] + ro_ref[...], 0.0)
    wl = wl_ref[...]
    wr = wr_ref[...]
    p_ref[...] = jnp.concatenate(
        [jnp.dot(he, wl, preferred_element_type=jnp.float32),
         jnp.dot(ho, wl, preferred_element_type=jnp.float32)], axis=1)
    rne_ref[...] = jnp.dot(he, wr, preferred_element_type=jnp.float32) + bl_ref[...]
    rno_ref[...] = jnp.dot(ho, wr, preferred_element_type=jnp.float32) + bl_ref[...]


def _final_body(ae0, ao0, ae1, ao1, ive, ivo, re_ref, ro_ref, o_ref):
    oe = (ae0[...] + ae1[...]) * ive[...] + re_ref[...]
    oo = (ao0[...] + ao1[...]) * ivo[...] + ro_ref[...]
    o_ref[...] = jnp.concatenate([oe, oo], axis=1)


def _pre(x128, wl, bl, wr):
    din, dout = wl.shape
    return pl.pallas_call(
        _pre_body,
        grid=(NH // 1000,),
        in_specs=[
            pl.BlockSpec((1000, din), lambda i: (i, 0)),
            pl.BlockSpec((din, dout), lambda i: (0, 0)),
            pl.BlockSpec((1, dout), lambda i: (0, 0)),
            pl.BlockSpec((din, dout), lambda i: (0, 0)),
        ],
        out_specs=[pl.BlockSpec((1000, dout), lambda i: (i, 0))] * 2,
        out_shape=[jax.ShapeDtypeStruct((NH, dout), jnp.float32)] * 2,
    )(x128, wl, bl, wr)


BRH = 1000


def _comb1(a128, c0e, c0o, c1e, c1o, rp, wl, bl, wr):
    din, dout = wl.shape
    cspec = pl.BlockSpec((BRH, 1), lambda i: (i, 0))
    ospec = pl.BlockSpec((BRH, dout), lambda i: (i, 0))
    return pl.pallas_call(
        _comb1_body,
        grid=(NH // BRH,),
        in_specs=[
            pl.BlockSpec((NC, BRH, 128), lambda i: (0, i, 0)),
            cspec, cspec, cspec, cspec,
            pl.BlockSpec((BRH, 128), lambda i: (i, 0)),
            pl.BlockSpec((din, dout), lambda i: (0, 0)),
            pl.BlockSpec((1, dout), lambda i: (0, 0)),
            pl.BlockSpec((din, dout), lambda i: (0, 0)),
        ],
        out_specs=[
            pl.BlockSpec((BRH, 2 * dout), lambda i: (i, 0)),
            ospec, ospec, cspec, cspec,
        ],
        out_shape=[
            jax.ShapeDtypeStruct((NH, 2 * dout), jnp.float32),
            jax.ShapeDtypeStruct((NH, dout), jnp.float32),
            jax.ShapeDtypeStruct((NH, dout), jnp.float32),
            jax.ShapeDtypeStruct((NH, 1), jnp.float32),
            jax.ShapeDtypeStruct((NH, 1), jnp.float32),
        ],
    )(a128, c0e, c0o, c1e, c1o, rp, wl, bl, wr)


def _comb(a128, ive, ivo, re, ro, wl, bl, wr):
    din, dout = wl.shape
    rspec = pl.BlockSpec((BRH, din), lambda i: (i, 0))
    cspec = pl.BlockSpec((BRH, 1), lambda i: (i, 0))
    ospec = pl.BlockSpec((BRH, dout), lambda i: (i, 0))
    return pl.pallas_call(
        _comb_body,
        grid=(NH // BRH,),
        in_specs=[
            pl.BlockSpec((NC, BRH, 128), lambda i: (0, i, 0)),
            cspec, cspec, rspec, rspec,
            pl.BlockSpec((din, dout), lambda i: (0, 0)),
            pl.BlockSpec((1, dout), lambda i: (0, 0)),
            pl.BlockSpec((din, dout), lambda i: (0, 0)),
        ],
        out_specs=[
            pl.BlockSpec((BRH, 2 * dout), lambda i: (i, 0)),
            ospec, ospec,
        ],
        out_shape=[
            jax.ShapeDtypeStruct((NH, 2 * dout), jnp.float32),
            jax.ShapeDtypeStruct((NH, dout), jnp.float32),
            jax.ShapeDtypeStruct((NH, dout), jnp.float32),
        ],
    )(a128, ive, ivo, re, ro, wl, bl, wr)


def _final(ae0, ao0, ae1, ao1, ive, ivo, re, ro):
    spec = pl.BlockSpec((NH, 1), lambda: (0, 0))
    return pl.pallas_call(
        _final_body,
        in_specs=[spec] * 8,
        out_specs=pl.BlockSpec((NH, 2), lambda: (0, 0)),
        out_shape=jax.ShapeDtypeStruct((NH, 2), jnp.float32),
    )(ae0, ao0, ae1, ao1, ive, ivo, re, ro)


def kernel(x, edge_index, Wl0, bl0, Wr0, Wl1, bl1, Wr1, Wl2, bl2, Wr2,
           Wl3, bl3, Wr3):
    ei3 = edge_index.reshape(2, NCHUNK, CH).transpose(1, 0, 2)
    z64 = jnp.zeros((CH, 64), jnp.float32)
    z16 = jnp.zeros((RPT, L), jnp.float32)
    iid = jnp.arange(RPT, dtype=jnp.int32).reshape(CPT, CH)

    x128 = x.reshape(NH, 256)
    zpad = jnp.zeros((128, 64), jnp.float32)
    wbl = jnp.concatenate(
        [jnp.concatenate([Wl0, zpad], 1), jnp.concatenate([zpad, Wl0], 1)], 0)
    wbr = jnp.concatenate(
        [jnp.concatenate([Wr0, zpad], 1), jnp.concatenate([zpad, Wr0], 1)], 0)
    blb = jnp.concatenate([bl0, bl0]).reshape(1, 128)

    p, rp = _pre(x128, wbl, blb, wbr)
    a, ac = _seg64c(ei3, p.reshape(N, 64), z64, z16, iid)
    acl = ac.reshape(NC, NP)
    c0e = acl[0, :NH].reshape(NH, 1)
    c0o = acl[0, HN:HN + NH].reshape(NH, 1)
    c1e = acl[1, :NH].reshape(NH, 1)
    c1o = acl[1, HN:HN + NH].reshape(NH, 1)
    p, re, ro, ive, ivo = _comb1(
        a.reshape(NC, HN, 128), c0e, c0o, c1e, c1o, rp,
        Wl1, bl1.reshape(1, -1), Wr1)
    for wl, bl, wr in ((Wl2, bl2, Wr2), (Wl3, bl3, Wr3)):
        a = _seg64(ei3, p.reshape(N, 64), z64)
        p, re, ro = _comb(a.reshape(NC, HN, 128), ive, ivo, re, ro,
                          wl, bl.reshape(1, -1), wr)

    p3 = jnp.pad(p.reshape(-1), (0, NP - N))
    a3 = _seg1(ei3, p3, z16, iid)
    a3l = a3.reshape(NC, NP)
    out = _final(a3l[0, :NH].reshape(NH, 1), a3l[0, HN:HN + NH].reshape(NH, 1),
                 a3l[1, :NH].reshape(NH, 1), a3l[1, HN:HN + NH].reshape(NH, 1),
                 ive, ivo, re, ro)
    return out.reshape(-1)


# speedup vs baseline: 21.0264x; 1.0900x over previous
import jax
import jax.numpy as jnp
from jax import lax
from jax.experimental import pallas as pl
from jax.experimental.pallas import tpu as pltpu
from jax.experimental.pallas import tpu_sc as plsc

N = 10000
E = 320000
NP = 10240
NC, NS, L = 2, 16, 16
NW = NC * NS
CH = 128
NCHUNK = E // CH
EPW = E // NW
RPT = NP // NS
CPW = NCHUNK // NW
K = 3
NWIN = CPW // K
NPAIR = NWIN // 2
CPT = RPT // CH
HN = NP // 2
NH = N // 2

_MESH = plsc.VectorSubcoreMesh(
    core_axis_name="c", subcore_axis_name="s", num_cores=NC, num_subcores=NS
)
_SC_PARAMS = pltpu.CompilerParams(
    needs_layout_passes=False, use_tc_tiling_on_sc=False
)


def _seg_body(with_cnt, *refs):
    if with_cnt:
        (ei_hbm, p_hbm, z_hbm, z16_hbm, ii_hbm, out_hbm, outc_hbm,
         eidx, ra, rb, cacc, iidx_v, acc_sh, red_sh, ga, gb, sa, sb) = refs
    else:
        (ei_hbm, p_hbm, z_hbm, out_hbm,
         eidx, ra, rb, acc_sh, ga, gb, sa, sb) = refs
    c = lax.axis_index("c")
    s = lax.axis_index("s")
    wid = s * NC + c
    row0 = wid * CPW

    pltpu.sync_copy(ei_hbm.at[pl.ds(row0, CPW)], eidx.at[pl.ds(0, CPW)])
    @pl.when(wid < 4)
    def _():
        xrow = NW * CPW + wid
        pltpu.sync_copy(ei_hbm.at[pl.ds(xrow, 1)], eidx.at[pl.ds(CPW, 1)])

    pltpu.sync_copy(z_hbm, ra.at[0])
    for k in range(CPT):
        pltpu.async_copy(ra.at[0], acc_sh.at[pl.ds(s * RPT + k * CH, CH), :], ga)
    for k in range(CPT):
        pltpu.make_async_copy(z_hbm, ra.at[0], ga).wait()
    if with_cnt:
        pltpu.sync_copy(z16_hbm, cacc)
        pltpu.sync_copy(ii_hbm, iidx_v)

        @pl.when(s == 0)
        def _():
            pltpu.sync_copy(cacc, red_sh)
    plsc.subcore_barrier()

    ones = jnp.ones((L,), jnp.float32)

    def _split_idx(d):
        idx = jnp.right_shift(d, 1) + jnp.bitwise_and(d, 1) * HN
        return [jnp.right_shift(idx, 4), jnp.bitwise_and(idx, 15)]

    def fire(w, buf, sem):
        for t in range(K):
            pltpu.async_copy(p_hbm.at[eidx.at[w * K + t, 0]], buf.at[t], sem)

    def drain(w, buf, sem):
        for t in range(K):
            pltpu.make_async_copy(
                p_hbm.at[eidx.at[w * K + t, 0]], buf.at[t], sem).wait()

    def scat(w, buf, sem):
        for t in range(K):
            pltpu.async_copy(buf.at[t], acc_sh.at[eidx.at[w * K + t, 1]],
                             sem, add=True)

    def scat_drain(w, buf, sem):
        for t in range(K):
            pltpu.make_async_copy(buf.at[t], acc_sh.at[eidx.at[w * K + t, 1]],
                                  sem).wait()

    def count(w):
        if not with_cnt:
            return
        for t in range(K):
            for u in range(CH // L):
                d = eidx[w * K + t, 1, pl.ds(u * L, L)]
                plsc.addupdate_scatter(cacc, _split_idx(d), ones)

    fire(0, ra, ga)

    @pl.loop(0, NPAIR)
    def _(j):
        wa = 2 * j

        @pl.when(j > 0)
        def _():
            scat_drain(wa - 1, rb, sb)

        fire(wa + 1, rb, gb)
        count(wa)
        drain(wa, ra, ga)
        scat(wa, ra, sa)
        count(wa + 1)
        drain(wa + 1, rb, gb)
        scat_drain(wa, ra, sa)

        @pl.when(j < NPAIR - 1)
        def _():
            fire(wa + 2, ra, ga)

        scat(wa + 1, rb, sb)

    scat_drain(2 * NPAIR - 1, rb, sb)

    @pl.when(wid < 4)
    def _():
        pltpu.async_copy(p_hbm.at[eidx.at[CPW, 0]], ra.at[0], ga).wait()
        pltpu.sync_copy(ra.at[0], acc_sh.at[eidx.at[CPW, 1]], add=True)
        if with_cnt:
            for u in range(CH // L):
                d = eidx[CPW, 1, pl.ds(u * L, L)]
                plsc.addupdate_scatter(cacc, _split_idx(d), ones)

    if with_cnt:
        for k in range(CPT):
            pltpu.sync_copy(cacc.at[pl.ds(k * CH, CH), :],
                            red_sh.at[iidx_v.at[k]], add=True)

    plsc.subcore_barrier()
    bufs = [ra.at[0], ra.at[1], ra.at[2], rb.at[0], rb.at[1]]
    for k in range(CPT):
        pltpu.async_copy(acc_sh.at[pl.ds(s * RPT + k * CH, CH), :], bufs[k], ga)
    for k in range(CPT):
        pltpu.make_async_copy(acc_sh.at[pl.ds(s * RPT + k * CH, CH), :],
                              bufs[k], ga).wait()
    for k in range(CPT):
        pltpu.async_copy(bufs[k], out_hbm.at[c, pl.ds(s * RPT + k * CH, CH), :], ga)
    for k in range(CPT):
        pltpu.make_async_copy(bufs[k],
                              out_hbm.at[c, pl.ds(s * RPT + k * CH, CH), :],
                              ga).wait()
    if with_cnt:
        @pl.when(s == 0)
        def _():
            pltpu.sync_copy(red_sh, cacc)
            pltpu.sync_copy(cacc, outc_hbm.at[c])


def _make_seg(with_cnt):
    width = 64
    out_type = jax.ShapeDtypeStruct((NC, NP, width), jnp.float32)
    scratch = [
        pltpu.VMEM((CPW + 1, 2, CH), jnp.int32),
        pltpu.VMEM((K, CH, width), jnp.float32),
        pltpu.VMEM((K, CH, width), jnp.float32),
    ]
    if with_cnt:
        out_type = [out_type, jax.ShapeDtypeStruct((NC, RPT, L), jnp.float32)]
        scratch += [
            pltpu.VMEM((RPT, L), jnp.float32),
            pltpu.VMEM((CPT, CH), jnp.int32),
        ]
    scratch.append(pltpu.VMEM_SHARED((NP, width), jnp.float32))
    if with_cnt:
        scratch.append(pltpu.VMEM_SHARED((RPT, L), jnp.float32))
    scratch += [pltpu.SemaphoreType.DMA] * 4

    def body(*refs):
        return _seg_body(with_cnt, *refs)

    return pl.kernel(
        body,
        out_type=out_type,
        mesh=_MESH,
        scratch_types=scratch,
        compiler_params=_SC_PARAMS,
    )


_seg64c = _make_seg(True)
_seg64 = _make_seg(False)


def _seg1_body(ei_hbm, p_hbm, z_hbm, ii_hbm, out_hbm,
               eidx, pv, acc_v, iidx_v, red_sh):
    c = lax.axis_index("c")
    s = lax.axis_index("s")
    wid = s * NC + c
    row0 = wid * CPW

    pltpu.sync_copy(z_hbm, acc_v)
    @pl.when(s == 0)
    def _():
        pltpu.sync_copy(acc_v, red_sh)
    pltpu.sync_copy(ii_hbm, iidx_v)
    pltpu.sync_copy(ei_hbm.at[pl.ds(row0, CPW)], eidx.at[pl.ds(0, CPW)])
    @pl.when(wid < 4)
    def _():
        pltpu.sync_copy(ei_hbm.at[pl.ds(NW * CPW + wid, 1)],
                        eidx.at[pl.ds(CPW, 1)])
    pltpu.sync_copy(p_hbm, pv)
    plsc.subcore_barrier()

    def do_chunk(i):
        for u in range(CH // L):
            sv = eidx[i, 0, pl.ds(u * L, L)]
            d = eidx[i, 1, pl.ds(u * L, L)]
            v = plsc.load_gather(pv, [sv])
            plsc.addupdate_scatter(
                acc_v, [jnp.right_shift(d, 4), jnp.bitwise_and(d, 15)], v
            )

    @pl.loop(0, CPW)
    def _(i):
        do_chunk(i)

    @pl.when(wid < 4)
    def _():
        do_chunk(CPW)

    for k in range(CPT):
        pltpu.sync_copy(acc_v.at[pl.ds(k * CH, CH), :],
                        red_sh.at[iidx_v.at[k]], add=True)
    plsc.subcore_barrier()

    @pl.when(s == 0)
    def _():
        pltpu.sync_copy(red_sh, acc_v)
        pltpu.sync_copy(acc_v, out_hbm.at[c])


_seg1 = pl.kernel(
    _seg1_body,
    out_type=jax.ShapeDtypeStruct((NC, RPT, L), jnp.float32),
    mesh=_MESH,
    scratch_types=[
        pltpu.VMEM((CPW + 1, 2, CH), jnp.int32),
        pltpu.VMEM((NP,), jnp.float32),
        pltpu.VMEM((RPT, L), jnp.float32),
        pltpu.VMEM((CPT, CH), jnp.int32),
        pltpu.VMEM_SHARED((RPT, L), jnp.float32),
    ],
    compiler_params=_SC_PARAMS,
)


BR = 2000


def _pre_body(h_ref, wl_ref, bl_ref, wr_ref, p_ref, r_ref):
    h = h_ref[...]
    p_ref[...] = jnp.dot(h, wl_ref[...], preferred_element_type=jnp.float32)
    r_ref[...] = (
        jnp.dot(h, wr_ref[...], preferred_element_type=jnp.float32) + bl_ref[...]
    )


def _comb1_body(a_ref, c0e, c0o, c1e, c1o, rp_ref,
                wl_ref, bl_ref, wr_ref,
                p_ref, rn_ref, iv_ref):
    s = a_ref[0] + a_ref[1]
    rp = rp_ref[...]
    inve = 1.0 / jnp.maximum(c0e[...] + c1e[...], 1.0)
    invo = 1.0 / jnp.maximum(c0o[...] + c1o[...], 1.0)
    he = jnp.maximum(s[:, :64] * inve + rp[:, :64], 0.0)
    ho = jnp.maximum(s[:, 64:] * invo + rp[:, 64:], 0.0)
    wl = wl_ref[...]
    wr = wr_ref[...]
    p_ref[...] = jnp.concatenate(
        [jnp.dot(he, wl, preferred_element_type=jnp.float32),
         jnp.dot(ho, wl, preferred_element_type=jnp.float32)], axis=1)
    bl = bl_ref[...]
    rn_ref[...] = jnp.concatenate(
        [jnp.dot(he, wr, preferred_element_type=jnp.float32) + bl,
         jnp.dot(ho, wr, preferred_element_type=jnp.float32) + bl], axis=1)
    iv_ref[...] = jnp.concatenate([inve, invo], axis=1)


def _comb_body(a_ref, iv_ref, rp_ref, wl_ref, bl_ref, wr_ref, p_ref, rn_ref):
    s = a_ref[0] + a_ref[1]
    iv = iv_ref[...]
    rp = rp_ref[...]
    he = jnp.maximum(s[:, :64] * iv[:, :1] + rp[:, :64], 0.0)
    ho = jnp.maximum(s[:, 64:] * iv[:, 1:] + rp[:, 64:], 0.0)
    wl = wl_ref[...]
    wr = wr_ref[...]
    p_ref[...] = jnp.concatenate(
        [jnp.dot(he, wl, preferred_element_type=jnp.float32),
         jnp.dot(ho, wl, preferred_element_type=jnp.float32)], axis=1)
    bl = bl_ref[...]
    rn_ref[...] = jnp.concatenate(
        [jnp.dot(he, wr, preferred_element_type=jnp.float32) + bl,
         jnp.dot(ho, wr, preferred_element_type=jnp.float32) + bl], axis=1)


def _final_body(a3_ref, iv_ref, r3_ref, o_ref):
    a = a3_ref[0] + a3_ref[1]
    o_ref[...] = a * iv_ref[...] + r3_ref[...]


def _pre(x128, wl, bl, wr):
    din, dout = wl.shape
    return pl.pallas_call(
        _pre_body,
        grid=(NH // 1000,),
        in_specs=[
            pl.BlockSpec((1000, din), lambda i: (i, 0)),
            pl.BlockSpec((din, dout), lambda i: (0, 0)),
            pl.BlockSpec((1, dout), lambda i: (0, 0)),
            pl.BlockSpec((din, dout), lambda i: (0, 0)),
        ],
        out_specs=[pl.BlockSpec((1000, dout), lambda i: (i, 0))] * 2,
        out_shape=[jax.ShapeDtypeStruct((NH, dout), jnp.float32)] * 2,
    )(x128, wl, bl, wr)


BRH = 1000


def _comb1(a128, c0e, c0o, c1e, c1o, rp, wl, bl, wr):
    din, dout = wl.shape
    cspec = pl.BlockSpec((BRH, 1), lambda i: (i, 0))
    ospec = pl.BlockSpec((BRH, dout), lambda i: (i, 0))
    return pl.pallas_call(
        _comb1_body,
        grid=(NH // BRH,),
        in_specs=[
            pl.BlockSpec((NC, BRH, 128), lambda i: (0, i, 0)),
            cspec, cspec, cspec, cspec,
            pl.BlockSpec((BRH, 128), lambda i: (i, 0)),
            pl.BlockSpec((din, dout), lambda i: (0, 0)),
            pl.BlockSpec((1, dout), lambda i: (0, 0)),
            pl.BlockSpec((din, dout), lambda i: (0, 0)),
        ],
        out_specs=[
            pl.BlockSpec((BRH, 2 * dout), lambda i: (i, 0)),
            pl.BlockSpec((BRH, 2 * dout), lambda i: (i, 0)),
            pl.BlockSpec((BRH, 2), lambda i: (i, 0)),
        ],
        out_shape=[
            jax.ShapeDtypeStruct((NH, 2 * dout), jnp.float32),
            jax.ShapeDtypeStruct((NH, 2 * dout), jnp.float32),
            jax.ShapeDtypeStruct((NH, 2), jnp.float32),
        ],
    )(a128, c0e, c0o, c1e, c1o, rp, wl, bl, wr)


def _comb(a128, iv, rp, wl, bl, wr):
    din, dout = wl.shape
    return pl.pallas_call(
        _comb_body,
        grid=(NH // BRH,),
        in_specs=[
            pl.BlockSpec((NC, BRH, 128), lambda i: (0, i, 0)),
            pl.BlockSpec((BRH, 2), lambda i: (i, 0)),
            pl.BlockSpec((BRH, 128), lambda i: (i, 0)),
            pl.BlockSpec((din, dout), lambda i: (0, 0)),
            pl.BlockSpec((1, dout), lambda i: (0, 0)),
            pl.BlockSpec((din, dout), lambda i: (0, 0)),
        ],
        out_specs=[pl.BlockSpec((BRH, 2 * dout), lambda i: (i, 0))] * 2,
        out_shape=[jax.ShapeDtypeStruct((NH, 2 * dout), jnp.float32)] * 2,
    )(a128, iv, rp, wl, bl, wr)


def _final(a3v, ivlin, r3lin):
    vspec = pl.BlockSpec((NP // 128, 128), lambda: (0, 0))
    return pl.pallas_call(
        _final_body,
        in_specs=[
            pl.BlockSpec((NC, NP // 128, 128), lambda: (0, 0, 0)),
            vspec, vspec,
        ],
        out_specs=vspec,
        out_shape=jax.ShapeDtypeStruct((NP // 128, 128), jnp.float32),
    )(a3v, ivlin, r3lin)


def kernel(x, edge_index, Wl0, bl0, Wr0, Wl1, bl1, Wr1, Wl2, bl2, Wr2,
           Wl3, bl3, Wr3):
    ei3 = edge_index.reshape(2, NCHUNK, CH).transpose(1, 0, 2)
    z64 = jnp.zeros((CH, 64), jnp.float32)
    z16 = jnp.zeros((RPT, L), jnp.float32)
    iid = jnp.arange(RPT, dtype=jnp.int32).reshape(CPT, CH)

    x128 = x.reshape(NH, 256)
    zpad = jnp.zeros((128, 64), jnp.float32)
    wbl = jnp.concatenate(
        [jnp.concatenate([Wl0, zpad], 1), jnp.concatenate([zpad, Wl0], 1)], 0)
    wbr = jnp.concatenate(
        [jnp.concatenate([Wr0, zpad], 1), jnp.concatenate([zpad, Wr0], 1)], 0)
    blb = jnp.concatenate([bl0, bl0]).reshape(1, 128)

    p, rp = _pre(x128, wbl, blb, wbr)
    a, ac = _seg64c(ei3, p.reshape(N, 64), z64, z16, iid)
    acl = ac.reshape(NC, NP)
    c0e = acl[0, :NH].reshape(NH, 1)
    c0o = acl[0, HN:HN + NH].reshape(NH, 1)
    c1e = acl[1, :NH].reshape(NH, 1)
    c1o = acl[1, HN:HN + NH].reshape(NH, 1)
    p, rp, iv = _comb1(
        a.reshape(NC, HN, 128), c0e, c0o, c1e, c1o, rp,
        Wl1, bl1.reshape(1, -1), Wr1)
    for wl, bl, wr in ((Wl2, bl2, Wr2), (Wl3, bl3, Wr3)):
        a = _seg64(ei3, p.reshape(N, 64), z64)
        p, rp = _comb(a.reshape(NC, HN, 128), iv, rp,
                      wl, bl.reshape(1, -1), wr)

    p3 = jnp.pad(p.reshape(-1), (0, NP - N))
    a3 = _seg1(ei3, p3, z16, iid)
    a3v = a3.reshape(NC, NP // 128, 128)
    ivlin = jnp.pad(iv.reshape(-1), (0, NP - N)).reshape(NP // 128, 128)
    r3lin = jnp.pad(rp.reshape(-1), (0, NP - N)).reshape(NP // 128, 128)
    out = _final(a3v, ivlin, r3lin)
    return out.reshape(-1)[:N]
